# Initial kernel scaffold; baseline (speedup 1.0000x reference)
import jax
import jax.numpy as jnp
from jax.experimental import pallas as pl


def kernel(h, edge_index, edge_mask, We, be, W1, b1, W2, b2):
    raise NotImplementedError("write your pallas kernel here")


# speedup vs baseline: 6.2188x; 6.2188x over previous
import functools

import jax
import jax.numpy as jnp
from jax import lax
from jax.experimental import pallas as pl
from jax.experimental.pallas import tpu as pltpu
from jax.experimental.pallas import tpu_sc as plsc

N = 10000
E = 320000
D = 128
DH = 64
NCLS = 40
CP = 48
NP = 10240
NC = 2
NS = 16
NW = NC * NS
EPS = E // NS
EPT = E // NW
BLK = 80
NBLK1 = EPS // BLK
NBLK2 = EPT // BLK
DCH = 2000
SL = NP // NS

_mesh = plsc.VectorSubcoreMesh(
    core_axis_name="c", subcore_axis_name="s", num_cores=NC, num_subcores=NS)

_GDN = lax.GatherDimensionNumbers(
    offset_dims=(), collapsed_slice_dims=(0,), start_index_map=(0,))


def _bcast_lane(v16, lane):
  idx = jnp.full((16, 1), lane, jnp.int32)
  return lax.gather(v16, idx, _GDN, (1,),
                    mode=lax.GatherScatterMode.PROMISE_IN_BOUNDS)


def _scale_rows(rows_ref, m16, gbase, nch):
  for e16 in range(16):
    mb = _bcast_lane(m16, e16)
    e = gbase + e16
    for c in range(nch):
      sl = rows_ref[e, pl.ds(c * 16, 16)]
      rows_ref[e, pl.ds(c * 16, 16)] = sl * mb


def _floop(n, body):
  lax.fori_loop(0, n, lambda i, c: (body(i), c)[1], 0)


@functools.partial(
    pl.kernel,
    out_type=[
        jax.ShapeDtypeStruct((E,), jnp.float32),
        jax.ShapeDtypeStruct((NP,), jnp.float32),
        jax.ShapeDtypeStruct((NC, NP, DH), jnp.float32),
    ],
    mesh=_mesh,
    compiler_params=pltpu.CompilerParams(needs_layout_passes=False, use_tc_tiling_on_sc=False),
    scratch_types=[
        pltpu.VMEM((NP,), jnp.float32),
        pltpu.VMEM((NP,), jnp.float32),
        pltpu.VMEM((NP,), jnp.float32),
        pltpu.VMEM((NP,), jnp.float32),
        pltpu.VMEM((DCH,), jnp.int32),
        pltpu.VMEM((DCH,), jnp.int32),
        pltpu.VMEM((BLK, DH), jnp.float32),
        pltpu.VMEM((BLK,), jnp.int32),
        pltpu.VMEM((BLK,), jnp.int32),
        pltpu.VMEM((BLK,), jnp.float32),
        pltpu.VMEM((BLK,), jnp.int32),
        pltpu.VMEM((BLK,), jnp.int32),
        pltpu.VMEM((BLK,), jnp.float32),
        pltpu.VMEM((SL,), jnp.float32),
        pltpu.VMEM((SL,), jnp.float32),
        pltpu.VMEM((SL,), jnp.float32),
        pltpu.VMEM((SL,), jnp.float32),
        pltpu.VMEM((48,), jnp.float32),
        pltpu.VMEM_SHARED((NS, 2, SL), jnp.float32),
        pltpu.VMEM_SHARED((2, NP), jnp.float32),
        pltpu.VMEM_SHARED((NP, DH), jnp.float32),
    ],
)
def _sc1(src_hbm, dst_hbm, em_hbm, a_hbm, b_hbm, g1x_hbm, scal_hbm,
         mask_hbm, degin_hbm, agg_hbm,
         ap, bp, dego, degi, dsrc, ddst, rows,
         srcin, dstin, emin, srcb, dstb, maskb,
         t640, t640b, acco, acci, scal, inbox, absh, aggsh):
  cid = lax.axis_index("c")
  sid = lax.axis_index("s")
  sl = sid * SL
  eb = sid * EPS
  z16 = jnp.zeros((16,), jnp.float32)
  ones16 = jnp.ones((16,), jnp.float32)
  off16 = jnp.full((16,), cid * NP, jnp.int32)

  pltpu.sync_copy(scal_hbm, scal)

  def zero_body(i):
    dego[pl.ds(i * 16, 16)] = z16
    degi[pl.ds(i * 16, 16)] = z16
  _floop(NP // 16, zero_body)

  def deg_chunk(ch):
    base = eb + ch * DCH
    pltpu.sync_copy(src_hbm.at[pl.ds(base, DCH)], dsrc)
    pltpu.sync_copy(dst_hbm.at[pl.ds(base, DCH)], ddst)

    def deg_body(i):
      s16 = dsrc[pl.ds(i * 16, 16)]
      d16 = ddst[pl.ds(i * 16, 16)]
      plsc.addupdate_scatter(dego, [s16], ones16)
      plsc.addupdate_scatter(degi, [d16], ones16)
    _floop(DCH // 16, deg_body)
  _floop(EPS // DCH, deg_chunk)

  def init_acc(q):
    acco[pl.ds(q * 16, 16)] = dego[pl.ds(sl + q * 16, 16)]
    acci[pl.ds(q * 16, 16)] = degi[pl.ds(sl + q * 16, 16)]
  _floop(SL // 16, init_acc)

  def ring_round(k):
    dest = lax.rem(sid + k, NS)
    srcslot = lax.rem(sid + NS - k, NS)
    plsc.subcore_barrier()
    pltpu.sync_copy(dego.at[pl.ds(dest * SL, SL)], inbox.at[sid, 0])
    pltpu.sync_copy(degi.at[pl.ds(dest * SL, SL)], inbox.at[sid, 1])
    plsc.subcore_barrier()
    pltpu.sync_copy(inbox.at[srcslot, 0], t640)
    pltpu.sync_copy(inbox.at[srcslot, 1], t640b)

    def acc_body(q):
      acco[pl.ds(q * 16, 16)] = acco[pl.ds(q * 16, 16)] + t640[pl.ds(q * 16, 16)]
      acci[pl.ds(q * 16, 16)] = acci[pl.ds(q * 16, 16)] + t640b[pl.ds(q * 16, 16)]
    _floop(SL // 16, acc_body)
  lax.fori_loop(1, NS, lambda k, c: (ring_round(k), c)[1], 0)

  pltpu.sync_copy(a_hbm.at[pl.ds(sl, SL)], t640)
  pltpu.sync_copy(b_hbm.at[pl.ds(sl, SL)], t640b)
  wa16 = scal[pl.ds(0, 16)]
  wb16 = scal[pl.ds(16, 16)]
  be16 = scal[pl.ds(32, 16)]

  def ab_body(q):
    t640[pl.ds(q * 16, 16)] = (t640[pl.ds(q * 16, 16)]
                               + wa16 * acco[pl.ds(q * 16, 16)] + be16)
    t640b[pl.ds(q * 16, 16)] = (t640b[pl.ds(q * 16, 16)]
                                + wb16 * acci[pl.ds(q * 16, 16)])
  _floop(SL // 16, ab_body)
  pltpu.sync_copy(t640, absh.at[0, pl.ds(sl, SL)])
  pltpu.sync_copy(t640b, absh.at[1, pl.ds(sl, SL)])

  @pl.when(cid == 0)
  def _():
    pltpu.sync_copy(acci, degin_hbm.at[pl.ds(sl, SL)])

  def zero_rows(r):
    for c in range(DH // 16):
      rows[r, pl.ds(c * 16, 16)] = z16
  _floop(BLK, zero_rows)
  for q in range(SL // BLK):
    pltpu.sync_copy(rows, aggsh.at[pl.ds(sl + q * BLK, BLK)])
  plsc.subcore_barrier()

  pltpu.sync_copy(absh.at[0], ap)
  pltpu.sync_copy(absh.at[1], bp)

  def blk_body(j):
    jb = eb + j * BLK
    pltpu.sync_copy(src_hbm.at[pl.ds(jb, BLK)], srcin)
    pltpu.sync_copy(dst_hbm.at[pl.ds(jb, BLK)], dstin)
    pltpu.sync_copy(em_hbm.at[pl.ds(jb, BLK)], emin)

    def grp_idx(g):
      gb = g * 16
      srcb[pl.ds(gb, 16)] = srcin[pl.ds(gb, 16)] + off16
      dstb[pl.ds(gb, 16)] = dstin[pl.ds(gb, 16)]
    _floop(BLK // 16, grp_idx)

    pltpu.sync_copy(g1x_hbm.at[srcb], rows)

    def grp_body(g):
      gb = g * 16
      s16 = srcin[pl.ds(gb, 16)]
      d16 = dstin[pl.ds(gb, 16)]
      em16 = emin[pl.ds(gb, 16)]
      sv = plsc.load_gather(ap, [s16]) + plsc.load_gather(bp, [d16])
      m16 = em16 / (1.0 + jnp.exp(-sv))
      maskb[pl.ds(gb, 16)] = m16
      _scale_rows(rows, m16, gb, DH // 16)
    _floop(BLK // 16, grp_body)

    @pl.when(cid == 0)
    def _():
      pltpu.sync_copy(maskb, mask_hbm.at[pl.ds(jb, BLK)])

    pltpu.sync_copy(rows, aggsh.at[dstb], add=True)
  _floop(NBLK1, blk_body)

  plsc.subcore_barrier()
  pltpu.sync_copy(aggsh.at[pl.ds(sl, SL)], agg_hbm.at[cid, pl.ds(sl, SL)])


@functools.partial(
    pl.kernel,
    out_type=[jax.ShapeDtypeStruct((NC, NP, CP), jnp.float32)],
    mesh=_mesh,
    compiler_params=pltpu.CompilerParams(needs_layout_passes=False, use_tc_tiling_on_sc=False),
    scratch_types=[
        pltpu.VMEM((BLK, CP), jnp.float32),
        pltpu.VMEM((BLK,), jnp.int32),
        pltpu.VMEM((BLK,), jnp.int32),
        pltpu.VMEM((BLK,), jnp.float32),
        pltpu.VMEM((BLK,), jnp.int32),
        pltpu.VMEM_SHARED((NP, CP), jnp.float32),
    ],
)
def _sc2(src_hbm, dst_hbm, m_hbm, g2_hbm, agg_hbm,
         rows, srcin, dstin, min_, dstb, aggsh):
  cid = lax.axis_index("c")
  sid = lax.axis_index("s")
  sl = sid * SL
  eb = (cid * NS + sid) * EPT
  z16 = jnp.zeros((16,), jnp.float32)

  def zero_rows(r):
    for c in range(CP // 16):
      rows[r, pl.ds(c * 16, 16)] = z16
  _floop(BLK, zero_rows)
  for q in range(SL // BLK):
    pltpu.sync_copy(rows, aggsh.at[pl.ds(sl + q * BLK, BLK)])
  plsc.subcore_barrier()

  def blk_body(j):
    jb = eb + j * BLK
    pltpu.sync_copy(src_hbm.at[pl.ds(jb, BLK)], srcin)
    pltpu.sync_copy(dst_hbm.at[pl.ds(jb, BLK)], dstin)
    pltpu.sync_copy(m_hbm.at[pl.ds(jb, BLK)], min_)
    pltpu.sync_copy(g2_hbm.at[srcin], rows)

    def grp_body(g):
      gb = g * 16
      dstb[pl.ds(gb, 16)] = dstin[pl.ds(gb, 16)]
      m16 = min_[pl.ds(gb, 16)]
      _scale_rows(rows, m16, gb, CP // 16)
    _floop(BLK // 16, grp_body)

    pltpu.sync_copy(rows, aggsh.at[dstb], add=True)
  _floop(NBLK2, blk_body)

  plsc.subcore_barrier()
  pltpu.sync_copy(aggsh.at[pl.ds(sl, SL)], agg_hbm.at[cid, pl.ds(sl, SL)])


def _tc0_body(x_ref, w_ref, o_ref):
  o_ref[...] = jnp.dot(x_ref[...], w_ref[...],
                       preferred_element_type=jnp.float32)


def _tc0(x, wc):
  return pl.pallas_call(
      _tc0_body,
      grid=(NP // SL,),
      in_specs=[pl.BlockSpec((SL, D), lambda i: (i, 0)),
                pl.BlockSpec((D, 256), lambda i: (0, 0))],
      out_specs=pl.BlockSpec((SL, 256), lambda i: (i, 0)),
      out_shape=jax.ShapeDtypeStruct((NP, 256), jnp.float32),
  )(x, wc)


def _tc1_body(p_ref, dg_ref, b1_ref, w2_ref, o_ref):
  recip = 1.0 / jnp.maximum(dg_ref[...], 1.0)
  pre = p_ref[...] * recip + b1_ref[...]
  h1 = jnp.maximum(pre, 0.0)
  o_ref[...] = jnp.dot(h1, w2_ref[...], preferred_element_type=jnp.float32)


def _tc1(p, dg, b1r, w2c):
  return pl.pallas_call(
      _tc1_body,
      grid=(NP // SL,),
      in_specs=[pl.BlockSpec((SL, D), lambda i: (i, 0)),
                pl.BlockSpec((SL, 1), lambda i: (i, 0)),
                pl.BlockSpec((1, D), lambda i: (0, 0)),
                pl.BlockSpec((D, CP), lambda i: (0, 0))],
      out_specs=pl.BlockSpec((SL, CP), lambda i: (i, 0)),
      out_shape=jax.ShapeDtypeStruct((NP, CP), jnp.float32),
  )(p, dg, b1r, w2c)


def _tc2_body(q0_ref, q1_ref, dg_ref, b2_ref, o_ref):
  recip = 1.0 / jnp.maximum(dg_ref[...], 1.0)
  o_ref[...] = (q0_ref[...] + q1_ref[...]) * recip + b2_ref[...]


def _tc2(q0, q1, dg, b2r):
  return pl.pallas_call(
      _tc2_body,
      grid=(NP // SL,),
      in_specs=[pl.BlockSpec((SL, CP), lambda i: (i, 0)),
                pl.BlockSpec((SL, CP), lambda i: (i, 0)),
                pl.BlockSpec((SL, 1), lambda i: (i, 0)),
                pl.BlockSpec((1, CP), lambda i: (0, 0))],
      out_specs=pl.BlockSpec((SL, CP), lambda i: (i, 0)),
      out_shape=jax.ShapeDtypeStruct((NP, CP), jnp.float32),
  )(q0, q1, dg, b2r)


def kernel(h, edge_index, edge_mask, We, be, W1, b1, W2, b2):
  f32 = jnp.float32
  src = edge_index[0]
  dst = edge_index[1]
  x = jnp.pad(h, ((0, NP - N), (0, 0)))
  wc = jnp.concatenate(
      [W1, We[:D], We[D:2 * D], jnp.zeros((D, 256 - D - 2), f32)], axis=1)
  g = _tc0(x, wc)
  a = g[:, D]
  b = g[:, D + 1]
  g1x = jnp.concatenate([g[:, :DH], g[:, DH:D]], axis=0)
  scal = jnp.concatenate([
      jnp.full((16,), We[2 * D, 0], f32),
      jnp.full((16,), We[2 * D + 1, 0], f32),
      jnp.full((16,), be[0], f32),
  ])
  mask, degin, agg1 = _sc1(src, dst, edge_mask, a, b, g1x, scal)
  degc = degin[:, None]
  p = jnp.concatenate([agg1[0], agg1[1]], axis=1)
  g2 = _tc1(p, degc, b1[None, :], jnp.pad(W2, ((0, 0), (0, CP - NCLS))))
  (agg2,) = _sc2(src, dst, mask, g2)
  out48 = _tc2(agg2[0], agg2[1], degc, jnp.pad(b2, (0, CP - NCLS))[None, :])
  return out48[:N, :NCLS]


# speedup vs baseline: 12.8996x; 2.0743x over previous
import functools

import jax
import jax.numpy as jnp
from jax import lax
from jax.experimental import pallas as pl
from jax.experimental.pallas import tpu as pltpu
from jax.experimental.pallas import tpu_sc as plsc

N = 10000
E = 320000
D = 128
DH = 64
NCLS = 40
CP = 48
NP = 10240
NC = 2
NS = 16
NW = NC * NS
EPS = E // NS
EPT = E // NW
BLK = 80
CHK = 4000
BPC = CHK // BLK
NCHK = EPS // CHK
NBLK2 = EPT // BLK
SL = NP // NS

_mesh = plsc.VectorSubcoreMesh(
    core_axis_name="c", subcore_axis_name="s", num_cores=NC, num_subcores=NS)

_GDN = lax.GatherDimensionNumbers(
    offset_dims=(), collapsed_slice_dims=(0,), start_index_map=(0,))


def _bcast_lane(v16, lane):
  idx = jnp.full((16, 1), lane, jnp.int32)
  return lax.gather(v16, idx, _GDN, (1,),
                    mode=lax.GatherScatterMode.PROMISE_IN_BOUNDS)


def _scale_rows(rows_ref, m16, gbase, nch):
  for e16 in range(16):
    mb = _bcast_lane(m16, e16)
    e = gbase + e16
    for c in range(nch):
      sl = rows_ref[e, pl.ds(c * 16, 16)]
      rows_ref[e, pl.ds(c * 16, 16)] = sl * mb


def _floop(n, body, lo=0):
  lax.fori_loop(lo, n, lambda i, c: (body(i), c)[1], 0)


@functools.partial(
    pl.kernel,
    out_type=[
        jax.ShapeDtypeStruct((E,), jnp.float32),
        jax.ShapeDtypeStruct((NP,), jnp.float32),
        jax.ShapeDtypeStruct((NC, NP, DH), jnp.float32),
    ],
    mesh=_mesh,
    compiler_params=pltpu.CompilerParams(
        needs_layout_passes=False, use_tc_tiling_on_sc=False),
    scratch_types=[
        pltpu.VMEM((NP,), jnp.float32),
        pltpu.VMEM((NP,), jnp.float32),
        pltpu.VMEM((NP,), jnp.float32),
        pltpu.VMEM((NP,), jnp.float32),
        pltpu.VMEM((CHK,), jnp.int32),
        pltpu.VMEM((CHK,), jnp.int32),
        pltpu.VMEM((CHK,), jnp.float32),
        pltpu.VMEM((CHK,), jnp.float32),
        pltpu.VMEM((BLK, DH), jnp.float32),
        pltpu.VMEM((BLK, DH), jnp.float32),
        pltpu.VMEM((BLK,), jnp.int32),
        pltpu.VMEM((BLK,), jnp.int32),
        pltpu.VMEM((BLK,), jnp.int32),
        pltpu.VMEM((BLK,), jnp.int32),
        pltpu.VMEM((SL,), jnp.float32),
        pltpu.VMEM((SL,), jnp.float32),
        pltpu.VMEM((SL,), jnp.float32),
        pltpu.VMEM((SL,), jnp.float32),
        pltpu.VMEM((48,), jnp.float32),
        pltpu.SemaphoreType.DMA,
        pltpu.SemaphoreType.DMA,
        pltpu.SemaphoreType.DMA,
        pltpu.SemaphoreType.DMA,
        pltpu.VMEM_SHARED((NS, 2, SL), jnp.float32),
        pltpu.VMEM_SHARED((2, NP), jnp.float32),
        pltpu.VMEM_SHARED((NP, DH), jnp.float32),
    ],
)
def _sc1(src_hbm, dst_hbm, em_hbm, a_hbm, b_hbm, g1x_hbm, scal_hbm,
         mask_hbm, degin_hbm, agg_hbm,
         ap, bp, dego, degi, csrc, cdst, cem, cmask,
         rows_a, rows_b, srcb_a, srcb_b, dstb_a, dstb_b,
         t640, t640b, acco, acci, scal,
         gsa, gsb, ssa, ssb, inbox, absh, aggsh):
  cid = lax.axis_index("c")
  sid = lax.axis_index("s")
  sl = sid * SL
  eb = sid * EPS
  z16 = jnp.zeros((16,), jnp.float32)
  ones16 = jnp.ones((16,), jnp.float32)
  off16 = jnp.full((16,), cid * NP, jnp.int32)

  pltpu.sync_copy(scal_hbm, scal)

  def zero_body(i):
    dego[pl.ds(i * 16, 16)] = z16
    degi[pl.ds(i * 16, 16)] = z16
  _floop(NP // 16, zero_body)

  for c in range(NCHK):
    base = eb + c * CHK
    pltpu.sync_copy(src_hbm.at[pl.ds(base, CHK)], csrc)
    pltpu.sync_copy(dst_hbm.at[pl.ds(base, CHK)], cdst)

    def deg_body(i):
      s16 = csrc[pl.ds(i * 16, 16)]
      d16 = cdst[pl.ds(i * 16, 16)]
      plsc.addupdate_scatter(dego, [s16], ones16)
      plsc.addupdate_scatter(degi, [d16], ones16)
    _floop(CHK // 16, deg_body)

  def init_acc(q):
    acco[pl.ds(q * 16, 16)] = dego[pl.ds(sl + q * 16, 16)]
    acci[pl.ds(q * 16, 16)] = degi[pl.ds(sl + q * 16, 16)]
  _floop(SL // 16, init_acc)

  def ring_round(k):
    dest = lax.rem(sid + k, NS)
    srcslot = lax.rem(sid + NS - k, NS)
    plsc.subcore_barrier()
    pltpu.sync_copy(dego.at[pl.ds(dest * SL, SL)], inbox.at[sid, 0])
    pltpu.sync_copy(degi.at[pl.ds(dest * SL, SL)], inbox.at[sid, 1])
    plsc.subcore_barrier()
    pltpu.sync_copy(inbox.at[srcslot, 0], t640)
    pltpu.sync_copy(inbox.at[srcslot, 1], t640b)

    def acc_body(q):
      acco[pl.ds(q * 16, 16)] = acco[pl.ds(q * 16, 16)] + t640[pl.ds(q * 16, 16)]
      acci[pl.ds(q * 16, 16)] = acci[pl.ds(q * 16, 16)] + t640b[pl.ds(q * 16, 16)]
    _floop(SL // 16, acc_body)
  _floop(NS, ring_round, lo=1)

  pltpu.sync_copy(a_hbm.at[pl.ds(sl, SL)], t640)
  pltpu.sync_copy(b_hbm.at[pl.ds(sl, SL)], t640b)
  wa16 = scal[pl.ds(0, 16)]
  wb16 = scal[pl.ds(16, 16)]
  be16 = scal[pl.ds(32, 16)]

  def ab_body(q):
    t640[pl.ds(q * 16, 16)] = (t640[pl.ds(q * 16, 16)]
                               + wa16 * acco[pl.ds(q * 16, 16)] + be16)
    t640b[pl.ds(q * 16, 16)] = (t640b[pl.ds(q * 16, 16)]
                                + wb16 * acci[pl.ds(q * 16, 16)])
  _floop(SL // 16, ab_body)
  pltpu.sync_copy(t640, absh.at[0, pl.ds(sl, SL)])
  pltpu.sync_copy(t640b, absh.at[1, pl.ds(sl, SL)])

  @pl.when(cid == 0)
  def _():
    pltpu.sync_copy(acci, degin_hbm.at[pl.ds(sl, SL)])

  def zero_rows(r):
    for c in range(DH // 16):
      rows_a[r, pl.ds(c * 16, 16)] = z16
  _floop(BLK, zero_rows)
  for q in range(SL // BLK):
    pltpu.sync_copy(rows_a, aggsh.at[pl.ds(sl + q * BLK, BLK)])
  plsc.subcore_barrier()

  pltpu.sync_copy(absh.at[0], ap)
  pltpu.sync_copy(absh.at[1], bp)

  def build_idx(srcb_x, dstb_x, boff):
    def gi(g):
      gb = g * 16
      srcb_x[pl.ds(gb, 16)] = csrc[pl.ds(boff + gb, 16)] + off16
      dstb_x[pl.ds(gb, 16)] = cdst[pl.ds(boff + gb, 16)]
    _floop(BLK // 16, gi)

  def compute_blk(rows_x, boff):
    def grp(g):
      gb = g * 16
      s16 = csrc[pl.ds(boff + gb, 16)]
      d16 = cdst[pl.ds(boff + gb, 16)]
      em16 = cem[pl.ds(boff + gb, 16)]
      sv = plsc.load_gather(ap, [s16]) + plsc.load_gather(bp, [d16])
      m16 = em16 / (1.0 + jnp.exp(-sv))
      cmask[pl.ds(boff + gb, 16)] = m16
      _scale_rows(rows_x, m16, gb, DH // 16)
    _floop(BLK // 16, grp)

  def g_start(srcb_x, rows_x, sem):
    pltpu.async_copy(g1x_hbm.at[srcb_x], rows_x, sem)

  def g_wait(srcb_x, rows_x, sem):
    pltpu.make_async_copy(g1x_hbm.at[srcb_x], rows_x, sem).wait()

  def s_start(rows_x, dstb_x, sem):
    pltpu.async_copy(rows_x, aggsh.at[dstb_x], sem, add=True)

  def s_wait(rows_x, dstb_x, sem):
    pltpu.make_async_copy(rows_x, aggsh.at[dstb_x], sem).wait()

  for c in range(NCHK):
    ce = eb + c * CHK
    pltpu.sync_copy(src_hbm.at[pl.ds(ce, CHK)], csrc)
    pltpu.sync_copy(dst_hbm.at[pl.ds(ce, CHK)], cdst)
    pltpu.sync_copy(em_hbm.at[pl.ds(ce, CHK)], cem)
    build_idx(srcb_a, dstb_a, 0)
    g_start(srcb_a, rows_a, gsa)

    def pair(t):
      b0 = t * (2 * BLK)
      b1 = b0 + BLK
      g_wait(srcb_a, rows_a, gsa)

      @pl.when(t > 0)
      def _():
        s_wait(rows_b, dstb_b, ssb)
      build_idx(srcb_b, dstb_b, b1)
      g_start(srcb_b, rows_b, gsb)
      compute_blk(rows_a, b0)
      s_start(rows_a, dstb_a, ssa)
      g_wait(srcb_b, rows_b, gsb)

      @pl.when(t < BPC // 2 - 1)
      def _():
        s_wait(rows_a, dstb_a, ssa)
        build_idx(srcb_a, dstb_a, b1 + BLK)
        g_start(srcb_a, rows_a, gsa)
      compute_blk(rows_b, b1)
      s_start(rows_b, dstb_b, ssb)
    _floop(BPC // 2, pair)

    s_wait(rows_a, dstb_a, ssa)
    s_wait(rows_b, dstb_b, ssb)

    @pl.when(cid == 0)
    def _():
      pltpu.sync_copy(cmask, mask_hbm.at[pl.ds(ce, CHK)])

  plsc.subcore_barrier()
  pltpu.sync_copy(aggsh.at[pl.ds(sl, SL)], agg_hbm.at[cid, pl.ds(sl, SL)])


@functools.partial(
    pl.kernel,
    out_type=[jax.ShapeDtypeStruct((NC, NP, CP), jnp.float32)],
    mesh=_mesh,
    compiler_params=pltpu.CompilerParams(
        needs_layout_passes=False, use_tc_tiling_on_sc=False),
    scratch_types=[
        pltpu.VMEM((EPT,), jnp.int32),
        pltpu.VMEM((EPT,), jnp.int32),
        pltpu.VMEM((EPT,), jnp.float32),
        pltpu.VMEM((BLK, CP), jnp.float32),
        pltpu.VMEM((BLK, CP), jnp.float32),
        pltpu.VMEM((BLK,), jnp.int32),
        pltpu.VMEM((BLK,), jnp.int32),
        pltpu.SemaphoreType.DMA,
        pltpu.SemaphoreType.DMA,
        pltpu.SemaphoreType.DMA,
        pltpu.SemaphoreType.DMA,
        pltpu.VMEM_SHARED((NP, CP), jnp.float32),
    ],
)
def _sc2(src_hbm, dst_hbm, m_hbm, g2_hbm, agg_hbm,
         src_sw, dst_sw, m_sw, rows_a, rows_b, dstb_a, dstb_b,
         gsa, gsb, ssa, ssb, aggsh):
  cid = lax.axis_index("c")
  sid = lax.axis_index("s")
  sl = sid * SL
  eb = (cid * NS + sid) * EPT
  z16 = jnp.zeros((16,), jnp.float32)

  def zero_rows(r):
    for c in range(CP // 16):
      rows_a[r, pl.ds(c * 16, 16)] = z16
  _floop(BLK, zero_rows)
  for q in range(SL // BLK):
    pltpu.sync_copy(rows_a, aggsh.at[pl.ds(sl + q * BLK, BLK)])
  plsc.subcore_barrier()

  pltpu.sync_copy(src_hbm.at[pl.ds(eb, EPT)], src_sw)
  pltpu.sync_copy(dst_hbm.at[pl.ds(eb, EPT)], dst_sw)
  pltpu.sync_copy(m_hbm.at[pl.ds(eb, EPT)], m_sw)

  def build_idx(dstb_x, boff):
    def gi(g):
      gb = g * 16
      dstb_x[pl.ds(gb, 16)] = dst_sw[pl.ds(boff + gb, 16)]
    _floop(BLK // 16, gi)

  def compute_blk(rows_x, boff):
    def grp(g):
      gb = g * 16
      m16 = m_sw[pl.ds(boff + gb, 16)]
      _scale_rows(rows_x, m16, gb, CP // 16)
    _floop(BLK // 16, grp)

  def g_start(boff, rows_x, sem):
    pltpu.async_copy(g2_hbm.at[src_sw.at[pl.ds(boff, BLK)]], rows_x, sem)

  def g_wait(boff, rows_x, sem):
    pltpu.make_async_copy(g2_hbm.at[src_sw.at[pl.ds(boff, BLK)]],
                          rows_x, sem).wait()

  def s_start(rows_x, dstb_x, sem):
    pltpu.async_copy(rows_x, aggsh.at[dstb_x], sem, add=True)

  def s_wait(rows_x, dstb_x, sem):
    pltpu.make_async_copy(rows_x, aggsh.at[dstb_x], sem).wait()

  build_idx(dstb_a, 0)
  g_start(0, rows_a, gsa)

  def pair(t):
    b0 = t * (2 * BLK)
    b1 = b0 + BLK
    g_wait(b0, rows_a, gsa)

    @pl.when(t > 0)
    def _():
      s_wait(rows_b, dstb_b, ssb)
    build_idx(dstb_b, b1)
    g_start(b1, rows_b, gsb)
    compute_blk(rows_a, b0)
    s_start(rows_a, dstb_a, ssa)
    g_wait(b1, rows_b, gsb)

    @pl.when(t < NBLK2 // 2)
    def _():
      s_wait(rows_a, dstb_a, ssa)
      build_idx(dstb_a, b1 + BLK)
      g_start(b1 + BLK, rows_a, gsa)
    compute_blk(rows_b, b1)
    s_start(rows_b, dstb_b, ssb)
  _floop(NBLK2 // 2, pair)

  tb = (NBLK2 - 1) * BLK
  g_wait(tb, rows_a, gsa)
  s_wait(rows_b, dstb_b, ssb)
  compute_blk(rows_a, tb)
  s_start(rows_a, dstb_a, ssa)
  s_wait(rows_a, dstb_a, ssa)

  plsc.subcore_barrier()
  pltpu.sync_copy(aggsh.at[pl.ds(sl, SL)], agg_hbm.at[cid, pl.ds(sl, SL)])


def _tc0_body(x_ref, w_ref, o_ref):
  o_ref[...] = jnp.dot(x_ref[...], w_ref[...],
                       preferred_element_type=jnp.float32)


def _tc0(x, wc):
  return pl.pallas_call(
      _tc0_body,
      grid=(NP // SL,),
      in_specs=[pl.BlockSpec((SL, D), lambda i: (i, 0)),
                pl.BlockSpec((D, 256), lambda i: (0, 0))],
      out_specs=pl.BlockSpec((SL, 256), lambda i: (i, 0)),
      out_shape=jax.ShapeDtypeStruct((NP, 256), jnp.float32),
  )(x, wc)


def _tc1_body(p_ref, dg_ref, b1_ref, w2_ref, o_ref):
  recip = 1.0 / jnp.maximum(dg_ref[...], 1.0)
  pre = p_ref[...] * recip + b1_ref[...]
  h1 = jnp.maximum(pre, 0.0)
  o_ref[...] = jnp.dot(h1, w2_ref[...], preferred_element_type=jnp.float32)


def _tc1(p, dg, b1r, w2c):
  return pl.pallas_call(
      _tc1_body,
      grid=(NP // SL,),
      in_specs=[pl.BlockSpec((SL, D), lambda i: (i, 0)),
                pl.BlockSpec((SL, 1), lambda i: (i, 0)),
                pl.BlockSpec((1, D), lambda i: (0, 0)),
                pl.BlockSpec((D, CP), lambda i: (0, 0))],
      out_specs=pl.BlockSpec((SL, CP), lambda i: (i, 0)),
      out_shape=jax.ShapeDtypeStruct((NP, CP), jnp.float32),
  )(p, dg, b1r, w2c)


def _tc2_body(q0_ref, q1_ref, dg_ref, b2_ref, o_ref):
  recip = 1.0 / jnp.maximum(dg_ref[...], 1.0)
  o_ref[...] = (q0_ref[...] + q1_ref[...]) * recip + b2_ref[...]


def _tc2(q0, q1, dg, b2r):
  return pl.pallas_call(
      _tc2_body,
      grid=(NP // SL,),
      in_specs=[pl.BlockSpec((SL, CP), lambda i: (i, 0)),
                pl.BlockSpec((SL, CP), lambda i: (i, 0)),
                pl.BlockSpec((SL, 1), lambda i: (i, 0)),
                pl.BlockSpec((1, CP), lambda i: (0, 0))],
      out_specs=pl.BlockSpec((SL, CP), lambda i: (i, 0)),
      out_shape=jax.ShapeDtypeStruct((NP, CP), jnp.float32),
  )(q0, q1, dg, b2r)


def kernel(h, edge_index, edge_mask, We, be, W1, b1, W2, b2):
  f32 = jnp.float32
  src = edge_index[0]
  dst = edge_index[1]
  x = jnp.pad(h, ((0, NP - N), (0, 0)))
  wc = jnp.concatenate(
      [W1, We[:D], We[D:2 * D], jnp.zeros((D, 256 - D - 2), f32)], axis=1)
  g = _tc0(x, wc)
  a = g[:, D]
  b = g[:, D + 1]
  g1x = jnp.concatenate([g[:, :DH], g[:, DH:D]], axis=0)
  scal = jnp.concatenate([
      jnp.full((16,), We[2 * D, 0], f32),
      jnp.full((16,), We[2 * D + 1, 0], f32),
      jnp.full((16,), be[0], f32),
  ])
  mask, degin, agg1 = _sc1(src, dst, edge_mask, a, b, g1x, scal)
  degc = degin[:, None]
  p = jnp.concatenate([agg1[0], agg1[1]], axis=1)
  g2 = _tc1(p, degc, b1[None, :], jnp.pad(W2, ((0, 0), (0, CP - NCLS))))
  (agg2,) = _sc2(src, dst, mask, g2)
  out48 = _tc2(agg2[0], agg2[1], degc, jnp.pad(b2, (0, CP - NCLS))[None, :])
  return out48[:N, :NCLS]


# speedup vs baseline: 18.8067x; 1.4579x over previous
import functools

import jax
import jax.numpy as jnp
from jax import lax
from jax.experimental import pallas as pl
from jax.experimental.pallas import tpu as pltpu
from jax.experimental.pallas import tpu_sc as plsc

N = 10000
E = 320000
D = 128
DH = 64
NCLS = 40
CP = 48
NP = 10240
NC = 2
NS = 16
NW = NC * NS
EPS = E // NS
EPT = E // NW
BLK = 80
CHK = 4000
BPC = CHK // BLK
NCHK = EPS // CHK
NBLK2 = EPT // BLK
SL = NP // NS

_mesh = plsc.VectorSubcoreMesh(
    core_axis_name="c", subcore_axis_name="s", num_cores=NC, num_subcores=NS)

_scp = pltpu.CompilerParams(
    needs_layout_passes=False, use_tc_tiling_on_sc=False)

_GDN = lax.GatherDimensionNumbers(
    offset_dims=(), collapsed_slice_dims=(0,), start_index_map=(0,))


def _bcast_lane(v16, lane):
  idx = jnp.full((16, 1), lane, jnp.int32)
  return lax.gather(v16, idx, _GDN, (1,),
                    mode=lax.GatherScatterMode.PROMISE_IN_BOUNDS)


def _scale_rows(rows_ref, m16, gbase, nch):
  for e16 in range(16):
    mb = _bcast_lane(m16, e16)
    e = gbase + e16
    for c in range(nch):
      sl = rows_ref[e, pl.ds(c * 16, 16)]
      rows_ref[e, pl.ds(c * 16, 16)] = sl * mb


def _floop(n, body, lo=0):
  lax.fori_loop(lo, n, lambda i, c: (body(i), c)[1], 0)


@functools.partial(
    pl.kernel,
    out_type=[jax.ShapeDtypeStruct((NW, 2, NP), jnp.float32)],
    mesh=_mesh,
    compiler_params=_scp,
    scratch_types=[
        pltpu.VMEM((NP,), jnp.float32),
        pltpu.VMEM((NP,), jnp.float32),
        pltpu.VMEM((EPT,), jnp.int32),
        pltpu.VMEM((EPT,), jnp.int32),
    ],
)
def _scdeg(src_hbm, dst_hbm, parts_hbm, dego, degi, esrc, edst):
  cid = lax.axis_index("c")
  sid = lax.axis_index("s")
  wid = cid * NS + sid
  z16 = jnp.zeros((16,), jnp.float32)
  ones16 = jnp.ones((16,), jnp.float32)

  def zero_body(i):
    dego[pl.ds(i * 16, 16)] = z16
    degi[pl.ds(i * 16, 16)] = z16
  _floop(NP // 16, zero_body)

  pltpu.sync_copy(src_hbm.at[pl.ds(wid * EPT, EPT)], esrc)
  pltpu.sync_copy(dst_hbm.at[pl.ds(wid * EPT, EPT)], edst)

  def deg_body(i):
    s16 = esrc[pl.ds(i * 16, 16)]
    d16 = edst[pl.ds(i * 16, 16)]
    plsc.addupdate_scatter(dego, [s16], ones16)
    plsc.addupdate_scatter(degi, [d16], ones16)
  _floop(EPT // 16, deg_body)

  pltpu.sync_copy(dego, parts_hbm.at[wid, 0])
  pltpu.sync_copy(degi, parts_hbm.at[wid, 1])


@functools.partial(
    pl.kernel,
    out_type=[
        jax.ShapeDtypeStruct((E,), jnp.float32),
        jax.ShapeDtypeStruct((NC, NP, DH), jnp.float32),
    ],
    mesh=_mesh,
    compiler_params=_scp,
    scratch_types=[
        pltpu.VMEM((NP,), jnp.float32),
        pltpu.VMEM((NP,), jnp.float32),
        pltpu.VMEM((CHK,), jnp.int32),
        pltpu.VMEM((CHK,), jnp.int32),
        pltpu.VMEM((CHK,), jnp.float32),
        pltpu.VMEM((CHK,), jnp.float32),
        pltpu.VMEM((BLK, DH), jnp.float32),
        pltpu.VMEM((BLK, DH), jnp.float32),
        pltpu.VMEM((BLK,), jnp.int32),
        pltpu.VMEM((BLK,), jnp.int32),
        pltpu.VMEM((BLK,), jnp.int32),
        pltpu.VMEM((BLK,), jnp.int32),
        pltpu.SemaphoreType.DMA,
        pltpu.SemaphoreType.DMA,
        pltpu.SemaphoreType.DMA,
        pltpu.SemaphoreType.DMA,
        pltpu.VMEM_SHARED((NP, DH), jnp.float32),
    ],
)
def _sc1(src_hbm, dst_hbm, em_hbm, ap_hbm, bp_hbm, g1x_hbm,
         mask_hbm, agg_hbm,
         ap, bp, csrc, cdst, cem, cmask,
         rows_a, rows_b, srcb_a, srcb_b, dstb_a, dstb_b,
         gsa, gsb, ssa, ssb, aggsh):
  cid = lax.axis_index("c")
  sid = lax.axis_index("s")
  sl = sid * SL
  eb = sid * EPS
  z16 = jnp.zeros((16,), jnp.float32)
  off16 = jnp.full((16,), cid * NP, jnp.int32)

  def zero_rows(r):
    for c in range(DH // 16):
      rows_a[r, pl.ds(c * 16, 16)] = z16
  _floop(BLK, zero_rows)
  for q in range(SL // BLK):
    pltpu.sync_copy(rows_a, aggsh.at[pl.ds(sl + q * BLK, BLK)])

  pltpu.sync_copy(ap_hbm, ap)
  pltpu.sync_copy(bp_hbm, bp)
  plsc.subcore_barrier()

  def build_idx(srcb_x, dstb_x, boff):
    for g in range(BLK // 16):
      gb = g * 16
      srcb_x[pl.ds(gb, 16)] = csrc[pl.ds(boff + gb, 16)] + off16
      dstb_x[pl.ds(gb, 16)] = cdst[pl.ds(boff + gb, 16)]

  def compute_blk(rows_x, boff):
    for g in range(BLK // 16):
      gb = g * 16
      s16 = csrc[pl.ds(boff + gb, 16)]
      d16 = cdst[pl.ds(boff + gb, 16)]
      em16 = cem[pl.ds(boff + gb, 16)]
      sv = plsc.load_gather(ap, [s16]) + plsc.load_gather(bp, [d16])
      m16 = em16 / (1.0 + jnp.exp(-sv))
      cmask[pl.ds(boff + gb, 16)] = m16
      _scale_rows(rows_x, m16, gb, DH // 16)

  def g_start(srcb_x, rows_x, sem):
    pltpu.async_copy(g1x_hbm.at[srcb_x], rows_x, sem)

  def g_wait(srcb_x, rows_x, sem):
    pltpu.make_async_copy(g1x_hbm.at[srcb_x], rows_x, sem).wait()

  def s_start(rows_x, dstb_x, sem):
    pltpu.async_copy(rows_x, aggsh.at[dstb_x], sem, add=True)

  def s_wait(rows_x, dstb_x, sem):
    pltpu.make_async_copy(rows_x, aggsh.at[dstb_x], sem).wait()

  for c in range(NCHK):
    ce = eb + c * CHK
    pltpu.sync_copy(src_hbm.at[pl.ds(ce, CHK)], csrc)
    pltpu.sync_copy(dst_hbm.at[pl.ds(ce, CHK)], cdst)
    pltpu.sync_copy(em_hbm.at[pl.ds(ce, CHK)], cem)
    build_idx(srcb_a, dstb_a, 0)
    g_start(srcb_a, rows_a, gsa)

    def pair(t):
      b0 = t * (2 * BLK)
      b1 = b0 + BLK
      g_wait(srcb_a, rows_a, gsa)

      @pl.when(t > 0)
      def _():
        s_wait(rows_b, dstb_b, ssb)
      build_idx(srcb_b, dstb_b, b1)
      g_start(srcb_b, rows_b, gsb)
      compute_blk(rows_a, b0)
      s_start(rows_a, dstb_a, ssa)
      g_wait(srcb_b, rows_b, gsb)

      @pl.when(t < BPC // 2 - 1)
      def _():
        s_wait(rows_a, dstb_a, ssa)
        build_idx(srcb_a, dstb_a, b1 + BLK)
        g_start(srcb_a, rows_a, gsa)
      compute_blk(rows_b, b1)
      s_start(rows_b, dstb_b, ssb)
    _floop(BPC // 2, pair)

    s_wait(rows_a, dstb_a, ssa)
    s_wait(rows_b, dstb_b, ssb)

    @pl.when(cid == 0)
    def _():
      pltpu.sync_copy(cmask, mask_hbm.at[pl.ds(ce, CHK)])

  plsc.subcore_barrier()
  pltpu.sync_copy(aggsh.at[pl.ds(sl, SL)], agg_hbm.at[cid, pl.ds(sl, SL)])


@functools.partial(
    pl.kernel,
    out_type=[jax.ShapeDtypeStruct((NC, NP, CP), jnp.float32)],
    mesh=_mesh,
    compiler_params=_scp,
    scratch_types=[
        pltpu.VMEM((EPT,), jnp.int32),
        pltpu.VMEM((EPT,), jnp.int32),
        pltpu.VMEM((EPT,), jnp.float32),
        pltpu.VMEM((BLK, CP), jnp.float32),
        pltpu.VMEM((BLK, CP), jnp.float32),
        pltpu.VMEM((BLK,), jnp.int32),
        pltpu.VMEM((BLK,), jnp.int32),
        pltpu.SemaphoreType.DMA,
        pltpu.SemaphoreType.DMA,
        pltpu.SemaphoreType.DMA,
        pltpu.SemaphoreType.DMA,
        pltpu.VMEM_SHARED((NP, CP), jnp.float32),
    ],
)
def _sc2(src_hbm, dst_hbm, m_hbm, g2_hbm, agg_hbm,
         src_sw, dst_sw, m_sw, rows_a, rows_b, dstb_a, dstb_b,
         gsa, gsb, ssa, ssb, aggsh):
  cid = lax.axis_index("c")
  sid = lax.axis_index("s")
  sl = sid * SL
  eb = (cid * NS + sid) * EPT
  z16 = jnp.zeros((16,), jnp.float32)

  def zero_rows(r):
    for c in range(CP // 16):
      rows_a[r, pl.ds(c * 16, 16)] = z16
  _floop(BLK, zero_rows)
  for q in range(SL // BLK):
    pltpu.sync_copy(rows_a, aggsh.at[pl.ds(sl + q * BLK, BLK)])
  plsc.subcore_barrier()

  pltpu.sync_copy(src_hbm.at[pl.ds(eb, EPT)], src_sw)
  pltpu.sync_copy(dst_hbm.at[pl.ds(eb, EPT)], dst_sw)
  pltpu.sync_copy(m_hbm.at[pl.ds(eb, EPT)], m_sw)

  def build_idx(dstb_x, boff):
    for g in range(BLK // 16):
      gb = g * 16
      dstb_x[pl.ds(gb, 16)] = dst_sw[pl.ds(boff + gb, 16)]

  def compute_blk(rows_x, boff):
    for g in range(BLK // 16):
      gb = g * 16
      m16 = m_sw[pl.ds(boff + gb, 16)]
      _scale_rows(rows_x, m16, gb, CP // 16)

  def g_start(boff, rows_x, sem):
    pltpu.async_copy(g2_hbm.at[src_sw.at[pl.ds(boff, BLK)]], rows_x, sem)

  def g_wait(boff, rows_x, sem):
    pltpu.make_async_copy(g2_hbm.at[src_sw.at[pl.ds(boff, BLK)]],
                          rows_x, sem).wait()

  def s_start(rows_x, dstb_x, sem):
    pltpu.async_copy(rows_x, aggsh.at[dstb_x], sem, add=True)

  def s_wait(rows_x, dstb_x, sem):
    pltpu.make_async_copy(rows_x, aggsh.at[dstb_x], sem).wait()

  build_idx(dstb_a, 0)
  g_start(0, rows_a, gsa)

  def pair(t):
    b0 = t * (2 * BLK)
    b1 = b0 + BLK
    g_wait(b0, rows_a, gsa)

    @pl.when(t > 0)
    def _():
      s_wait(rows_b, dstb_b, ssb)
    build_idx(dstb_b, b1)
    g_start(b1, rows_b, gsb)
    compute_blk(rows_a, b0)
    s_start(rows_a, dstb_a, ssa)
    g_wait(b1, rows_b, gsb)

    @pl.when(t < NBLK2 // 2)
    def _():
      s_wait(rows_a, dstb_a, ssa)
      build_idx(dstb_a, b1 + BLK)
      g_start(b1 + BLK, rows_a, gsa)
    compute_blk(rows_b, b1)
    s_start(rows_b, dstb_b, ssb)
  _floop(NBLK2 // 2, pair)

  tb = (NBLK2 - 1) * BLK
  g_wait(tb, rows_a, gsa)
  s_wait(rows_b, dstb_b, ssb)
  compute_blk(rows_a, tb)
  s_start(rows_a, dstb_a, ssa)
  s_wait(rows_a, dstb_a, ssa)

  plsc.subcore_barrier()
  pltpu.sync_copy(aggsh.at[pl.ds(sl, SL)], agg_hbm.at[cid, pl.ds(sl, SL)])


def _tc0_body(x_ref, w1_ref, wab_ref, o1_ref, oab_ref):
  x = x_ref[...]
  o1_ref[0] = jnp.dot(x, w1_ref[0], preferred_element_type=jnp.float32)
  oab_ref[...] = jnp.dot(x, wab_ref[...],
                         preferred_element_type=jnp.float32)


def _tc0(x, w1, wab):
  return pl.pallas_call(
      _tc0_body,
      grid=(NP // SL, 2),
      in_specs=[pl.BlockSpec((SL, D), lambda i, j: (i, 0)),
                pl.BlockSpec((1, D, DH), lambda i, j: (j, 0, 0)),
                pl.BlockSpec((D, 8), lambda i, j: (0, 0))],
      out_specs=[pl.BlockSpec((1, SL, DH), lambda i, j: (j, i, 0)),
                 pl.BlockSpec((SL, 8), lambda i, j: (i, 0))],
      out_shape=[jax.ShapeDtypeStruct((2, NP, DH), jnp.float32),
                 jax.ShapeDtypeStruct((NP, 8), jnp.float32)],
  )(x, w1, wab)


def _tcdeg_body(parts_ref, a_ref, b_ref, sc_ref, o_ref):
  r = jnp.sum(parts_ref[...], axis=0)
  dego = r[0:1]
  degi = r[1:2]
  wa = sc_ref[0, 0]
  wb = sc_ref[0, 1]
  be = sc_ref[0, 2]
  apv = a_ref[...] + wa * dego + be
  bpv = b_ref[...] + wb * degi
  o_ref[...] = jnp.concatenate([apv, bpv, degi], axis=0)


def _tcdeg(parts, a2, b2, scalv):
  return pl.pallas_call(
      _tcdeg_body,
      grid=(1,),
      in_specs=[pl.BlockSpec((NW, 2, NP), lambda i: (0, 0, 0)),
                pl.BlockSpec((1, NP), lambda i: (0, 0)),
                pl.BlockSpec((1, NP), lambda i: (0, 0)),
                pl.BlockSpec((1, 128), lambda i: (0, 0))],
      out_specs=pl.BlockSpec((3, NP), lambda i: (0, 0)),
      out_shape=jax.ShapeDtypeStruct((3, NP), jnp.float32),
  )(parts, a2, b2, scalv)


def _tc1_body(p0_ref, p1_ref, dg_ref, b1_ref, w2_ref, o_ref):
  recip = 1.0 / jnp.maximum(dg_ref[...], 1.0)
  p = jnp.concatenate([p0_ref[0], p1_ref[0]], axis=1)
  pre = p * recip + b1_ref[...]
  h1 = jnp.maximum(pre, 0.0)
  o_ref[...] = jnp.dot(h1, w2_ref[...], preferred_element_type=jnp.float32)


def _tc1(agg1, dg, b1r, w2c):
  return pl.pallas_call(
      _tc1_body,
      grid=(NP // SL,),
      in_specs=[pl.BlockSpec((1, SL, DH), lambda i: (0, i, 0)),
                pl.BlockSpec((1, SL, DH), lambda i: (1, i, 0)),
                pl.BlockSpec((SL, 1), lambda i: (i, 0)),
                pl.BlockSpec((1, D), lambda i: (0, 0)),
                pl.BlockSpec((D, CP), lambda i: (0, 0))],
      out_specs=pl.BlockSpec((SL, CP), lambda i: (i, 0)),
      out_shape=jax.ShapeDtypeStruct((NP, CP), jnp.float32),
  )(agg1, agg1, dg, b1r, w2c)


def _tc2_body(q0_ref, q1_ref, dg_ref, b2_ref, o_ref):
  recip = 1.0 / jnp.maximum(dg_ref[...], 1.0)
  o_ref[...] = ((q0_ref[0] + q1_ref[0]) * recip + b2_ref[...])[:, :NCLS]


def _tc2(agg2, dg, b2r):
  nb = 10
  rb = N // nb
  return pl.pallas_call(
      _tc2_body,
      grid=(nb,),
      in_specs=[pl.BlockSpec((1, rb, CP), lambda i: (0, i, 0)),
                pl.BlockSpec((1, rb, CP), lambda i: (1, i, 0)),
                pl.BlockSpec((rb, 1), lambda i: (i, 0)),
                pl.BlockSpec((1, CP), lambda i: (0, 0))],
      out_specs=pl.BlockSpec((rb, NCLS), lambda i: (i, 0)),
      out_shape=jax.ShapeDtypeStruct((N, NCLS), jnp.float32),
  )(agg2, agg2, dg, b2r)


def kernel(h, edge_index, edge_mask, We, be, W1, b1, W2, b2):
  f32 = jnp.float32
  src = edge_index[0]
  dst = edge_index[1]
  x = jnp.pad(h, ((0, NP - N), (0, 0)))
  wab = jnp.concatenate(
      [We[:D], We[D:2 * D], jnp.zeros((D, 6), f32)], axis=1)
  w1s = jnp.stack([W1[:, :DH], W1[:, DH:]], axis=0)
  g1s, oab = _tc0(x, w1s, wab)
  g1x = g1s.reshape(2 * NP, DH)
  a2 = oab[:, 0].reshape(1, NP)
  b2c = oab[:, 1].reshape(1, NP)
  scalv = jnp.zeros((1, 128), f32)
  scalv = scalv.at[0, 0].set(We[2 * D, 0])
  scalv = scalv.at[0, 1].set(We[2 * D + 1, 0])
  scalv = scalv.at[0, 2].set(be[0])
  (parts,) = _scdeg(src, dst)
  o3 = _tcdeg(parts, a2, b2c, scalv)
  apv = o3[0]
  bpv = o3[1]
  degc = o3[2][:, None]
  mask, agg1 = _sc1(src, dst, edge_mask, apv, bpv, g1x)
  g2 = _tc1(agg1, degc, b1[None, :], jnp.pad(W2, ((0, 0), (0, CP - NCLS))))
  (agg2,) = _sc2(src, dst, mask, g2)
  return _tc2(agg2, degc, jnp.pad(b2, (0, CP - NCLS))[None, :])


# speedup vs baseline: 19.6993x; 1.0475x over previous
import functools

import jax
import jax.numpy as jnp
from jax import lax
from jax.experimental import pallas as pl
from jax.experimental.pallas import tpu as pltpu
from jax.experimental.pallas import tpu_sc as plsc

N = 10000
E = 320000
D = 128
DH = 64
NCLS = 40
CP = 48
NP = 10240
NC = 2
NS = 16
NW = NC * NS
EPS = E // NS
EPT = E // NW
BLK = 80
CHK = 4000
BPC = CHK // BLK
NCHK = EPS // CHK
NBLK2 = EPT // BLK
SL = NP // NS

_mesh = plsc.VectorSubcoreMesh(
    core_axis_name="c", subcore_axis_name="s", num_cores=NC, num_subcores=NS)

_scp = pltpu.CompilerParams(
    needs_layout_passes=False, use_tc_tiling_on_sc=False)

_GDN = lax.GatherDimensionNumbers(
    offset_dims=(), collapsed_slice_dims=(0,), start_index_map=(0,))


def _bcast_lane(v16, lane):
  idx = jnp.full((16, 1), lane, jnp.int32)
  return lax.gather(v16, idx, _GDN, (1,),
                    mode=lax.GatherScatterMode.PROMISE_IN_BOUNDS)


def _scale_rows(rows_ref, m16, gbase, nch):
  for e16 in range(16):
    mb = _bcast_lane(m16, e16)
    e = gbase + e16
    for c in range(nch):
      sl = rows_ref[e, pl.ds(c * 16, 16)]
      rows_ref[e, pl.ds(c * 16, 16)] = sl * mb


def _floop(n, body, lo=0):
  lax.fori_loop(lo, n, lambda i, c: (body(i), c)[1], 0)


@functools.partial(
    pl.kernel,
    out_type=[jax.ShapeDtypeStruct((NW, 2, NP), jnp.float32)],
    mesh=_mesh,
    compiler_params=_scp,
    scratch_types=[
        pltpu.VMEM((NP,), jnp.float32),
        pltpu.VMEM((NP,), jnp.float32),
        pltpu.VMEM((EPT,), jnp.int32),
        pltpu.VMEM((EPT,), jnp.int32),
    ],
)
def _scdeg(src_hbm, dst_hbm, parts_hbm, dego, degi, esrc, edst):
  cid = lax.axis_index("c")
  sid = lax.axis_index("s")
  wid = cid * NS + sid
  z16 = jnp.zeros((16,), jnp.float32)
  ones16 = jnp.ones((16,), jnp.float32)

  def zero_body(i):
    dego[pl.ds(i * 16, 16)] = z16
    degi[pl.ds(i * 16, 16)] = z16
  _floop(NP // 16, zero_body)

  pltpu.sync_copy(src_hbm.at[pl.ds(wid * EPT, EPT)], esrc)
  pltpu.sync_copy(dst_hbm.at[pl.ds(wid * EPT, EPT)], edst)

  def deg_body(i):
    s16 = esrc[pl.ds(i * 16, 16)]
    d16 = edst[pl.ds(i * 16, 16)]
    plsc.addupdate_scatter(dego, [s16], ones16)
    plsc.addupdate_scatter(degi, [d16], ones16)
  _floop(EPT // 16, deg_body)

  pltpu.sync_copy(dego, parts_hbm.at[wid, 0])
  pltpu.sync_copy(degi, parts_hbm.at[wid, 1])


@functools.partial(
    pl.kernel,
    out_type=[
        jax.ShapeDtypeStruct((E,), jnp.float32),
        jax.ShapeDtypeStruct((NC, NP, DH), jnp.float32),
    ],
    mesh=_mesh,
    compiler_params=_scp,
    scratch_types=[
        pltpu.VMEM((NP,), jnp.float32),
        pltpu.VMEM((NP,), jnp.float32),
        pltpu.VMEM((CHK,), jnp.int32),
        pltpu.VMEM((CHK,), jnp.int32),
        pltpu.VMEM((CHK,), jnp.float32),
        pltpu.VMEM((CHK,), jnp.float32),
        pltpu.VMEM((CHK,), jnp.int32),
        pltpu.VMEM((CHK,), jnp.int32),
        pltpu.VMEM((CHK,), jnp.float32),
        pltpu.VMEM((BLK, DH), jnp.float32),
        pltpu.VMEM((BLK, DH), jnp.float32),
        pltpu.VMEM((BLK,), jnp.int32),
        pltpu.VMEM((BLK,), jnp.int32),
        pltpu.VMEM((BLK,), jnp.int32),
        pltpu.VMEM((BLK,), jnp.int32),
        pltpu.SemaphoreType.DMA,
        pltpu.SemaphoreType.DMA,
        pltpu.SemaphoreType.DMA,
        pltpu.SemaphoreType.DMA,
        pltpu.SemaphoreType.DMA,
        pltpu.VMEM_SHARED((NP, DH), jnp.float32),
    ],
)
def _sc1(src_hbm, dst_hbm, em_hbm, ap_hbm, bp_hbm, g1x_hbm,
         mask_hbm, agg_hbm,
         ap, bp, csrc, cdst, cem, cmask, csrc2, cdst2, cem2,
         rows_a, rows_b, srcb_a, srcb_b, dstb_a, dstb_b,
         gsa, gsb, ssa, ssb, csem, aggsh):
  cid = lax.axis_index("c")
  sid = lax.axis_index("s")
  sl = sid * SL
  eb = sid * EPS
  z16 = jnp.zeros((16,), jnp.float32)
  off16 = jnp.full((16,), cid * NP, jnp.int32)

  def zero_rows(r):
    for c in range(DH // 16):
      rows_a[r, pl.ds(c * 16, 16)] = z16
  _floop(BLK, zero_rows)
  for q in range(SL // BLK):
    pltpu.sync_copy(rows_a, aggsh.at[pl.ds(sl + q * BLK, BLK)])

  pltpu.sync_copy(ap_hbm, ap)
  pltpu.sync_copy(bp_hbm, bp)
  plsc.subcore_barrier()

  def build_idx(bufs, srcb_x, dstb_x, boff):
    for g in range(BLK // 16):
      gb = g * 16
      srcb_x[pl.ds(gb, 16)] = bufs[0][pl.ds(boff + gb, 16)] + off16
      dstb_x[pl.ds(gb, 16)] = bufs[1][pl.ds(boff + gb, 16)]

  def compute_blk(bufs, rows_x, boff):
    for g in range(BLK // 16):
      gb = g * 16
      s16 = bufs[0][pl.ds(boff + gb, 16)]
      d16 = bufs[1][pl.ds(boff + gb, 16)]
      em16 = bufs[2][pl.ds(boff + gb, 16)]
      sv = plsc.load_gather(ap, [s16]) + plsc.load_gather(bp, [d16])
      m16 = em16 / (1.0 + jnp.exp(-sv))
      cmask[pl.ds(boff + gb, 16)] = m16
      _scale_rows(rows_x, m16, gb, DH // 16)

  def g_start(srcb_x, rows_x, sem):
    pltpu.async_copy(g1x_hbm.at[srcb_x], rows_x, sem)

  def g_wait(srcb_x, rows_x, sem):
    pltpu.make_async_copy(g1x_hbm.at[srcb_x], rows_x, sem).wait()

  def s_start(rows_x, dstb_x, sem):
    pltpu.async_copy(rows_x, aggsh.at[dstb_x], sem, add=True)

  def s_wait(rows_x, dstb_x, sem):
    pltpu.make_async_copy(rows_x, aggsh.at[dstb_x], sem).wait()

  sets = ((csrc, cdst, cem), (csrc2, cdst2, cem2))

  def chunk_load(c, bufs):
    ce = eb + c * CHK
    pltpu.async_copy(src_hbm.at[pl.ds(ce, CHK)], bufs[0], csem)
    pltpu.async_copy(dst_hbm.at[pl.ds(ce, CHK)], bufs[1], csem)
    pltpu.async_copy(em_hbm.at[pl.ds(ce, CHK)], bufs[2], csem)

  def chunk_wait(c, bufs):
    ce = eb + c * CHK
    pltpu.make_async_copy(src_hbm.at[pl.ds(ce, CHK)], bufs[0], csem).wait()
    pltpu.make_async_copy(dst_hbm.at[pl.ds(ce, CHK)], bufs[1], csem).wait()
    pltpu.make_async_copy(em_hbm.at[pl.ds(ce, CHK)], bufs[2], csem).wait()

  chunk_load(0, sets[0])
  for c in range(NCHK):
    ce = eb + c * CHK
    bufs = sets[c % 2]
    chunk_wait(c, bufs)
    if c + 1 < NCHK:
      chunk_load(c + 1, sets[(c + 1) % 2])
    build_idx(bufs, srcb_a, dstb_a, 0)
    g_start(srcb_a, rows_a, gsa)

    def pair(t):
      b0 = t * (2 * BLK)
      b1 = b0 + BLK
      g_wait(srcb_a, rows_a, gsa)

      @pl.when(t > 0)
      def _():
        s_wait(rows_b, dstb_b, ssb)
      build_idx(bufs, srcb_b, dstb_b, b1)
      g_start(srcb_b, rows_b, gsb)
      compute_blk(bufs, rows_a, b0)
      s_start(rows_a, dstb_a, ssa)
      g_wait(srcb_b, rows_b, gsb)

      @pl.when(t < BPC // 2 - 1)
      def _():
        s_wait(rows_a, dstb_a, ssa)
        build_idx(bufs, srcb_a, dstb_a, b1 + BLK)
        g_start(srcb_a, rows_a, gsa)
      compute_blk(bufs, rows_b, b1)
      s_start(rows_b, dstb_b, ssb)
    _floop(BPC // 2, pair)

    s_wait(rows_a, dstb_a, ssa)
    s_wait(rows_b, dstb_b, ssb)

    @pl.when(cid == 0)
    def _():
      pltpu.sync_copy(cmask, mask_hbm.at[pl.ds(ce, CHK)])

  plsc.subcore_barrier()
  pltpu.sync_copy(aggsh.at[pl.ds(sl, SL)], agg_hbm.at[cid, pl.ds(sl, SL)])


@functools.partial(
    pl.kernel,
    out_type=[jax.ShapeDtypeStruct((NC, NP, CP), jnp.float32)],
    mesh=_mesh,
    compiler_params=_scp,
    scratch_types=[
        pltpu.VMEM((EPT,), jnp.int32),
        pltpu.VMEM((EPT,), jnp.int32),
        pltpu.VMEM((EPT,), jnp.float32),
        pltpu.VMEM((BLK, CP), jnp.float32),
        pltpu.VMEM((BLK, CP), jnp.float32),
        pltpu.VMEM((BLK,), jnp.int32),
        pltpu.VMEM((BLK,), jnp.int32),
        pltpu.SemaphoreType.DMA,
        pltpu.SemaphoreType.DMA,
        pltpu.SemaphoreType.DMA,
        pltpu.SemaphoreType.DMA,
        pltpu.SemaphoreType.DMA,
        pltpu.VMEM_SHARED((NP, CP), jnp.float32),
    ],
)
def _sc2(src_hbm, dst_hbm, m_hbm, g2_hbm, agg_hbm,
         src_sw, dst_sw, m_sw, rows_a, rows_b, dstb_a, dstb_b,
         gsa, gsb, ssa, ssb, csem, aggsh):
  cid = lax.axis_index("c")
  sid = lax.axis_index("s")
  sl = sid * SL
  eb = (cid * NS + sid) * EPT
  z16 = jnp.zeros((16,), jnp.float32)

  pltpu.async_copy(src_hbm.at[pl.ds(eb, EPT)], src_sw, csem)
  pltpu.async_copy(dst_hbm.at[pl.ds(eb, EPT)], dst_sw, csem)
  pltpu.async_copy(m_hbm.at[pl.ds(eb, EPT)], m_sw, csem)

  def zero_rows(r):
    for c in range(CP // 16):
      rows_a[r, pl.ds(c * 16, 16)] = z16
  _floop(BLK, zero_rows)
  for q in range(SL // BLK):
    pltpu.sync_copy(rows_a, aggsh.at[pl.ds(sl + q * BLK, BLK)])
  plsc.subcore_barrier()

  pltpu.make_async_copy(src_hbm.at[pl.ds(eb, EPT)], src_sw, csem).wait()
  pltpu.make_async_copy(dst_hbm.at[pl.ds(eb, EPT)], dst_sw, csem).wait()
  pltpu.make_async_copy(m_hbm.at[pl.ds(eb, EPT)], m_sw, csem).wait()

  def build_idx(dstb_x, boff):
    for g in range(BLK // 16):
      gb = g * 16
      dstb_x[pl.ds(gb, 16)] = dst_sw[pl.ds(boff + gb, 16)]

  def compute_blk(rows_x, boff):
    for g in range(BLK // 16):
      gb = g * 16
      m16 = m_sw[pl.ds(boff + gb, 16)]
      _scale_rows(rows_x, m16, gb, CP // 16)

  def g_start(boff, rows_x, sem):
    pltpu.async_copy(g2_hbm.at[src_sw.at[pl.ds(boff, BLK)]], rows_x, sem)

  def g_wait(boff, rows_x, sem):
    pltpu.make_async_copy(g2_hbm.at[src_sw.at[pl.ds(boff, BLK)]],
                          rows_x, sem).wait()

  def s_start(rows_x, dstb_x, sem):
    pltpu.async_copy(rows_x, aggsh.at[dstb_x], sem, add=True)

  def s_wait(rows_x, dstb_x, sem):
    pltpu.make_async_copy(rows_x, aggsh.at[dstb_x], sem).wait()

  build_idx(dstb_a, 0)
  g_start(0, rows_a, gsa)

  def pair(t):
    b0 = t * (2 * BLK)
    b1 = b0 + BLK
    g_wait(b0, rows_a, gsa)

    @pl.when(t > 0)
    def _():
      s_wait(rows_b, dstb_b, ssb)
    build_idx(dstb_b, b1)
    g_start(b1, rows_b, gsb)
    compute_blk(rows_a, b0)
    s_start(rows_a, dstb_a, ssa)
    g_wait(b1, rows_b, gsb)

    @pl.when(t < NBLK2 // 2)
    def _():
      s_wait(rows_a, dstb_a, ssa)
      build_idx(dstb_a, b1 + BLK)
      g_start(b1 + BLK, rows_a, gsa)
    compute_blk(rows_b, b1)
    s_start(rows_b, dstb_b, ssb)
  _floop(NBLK2 // 2, pair)

  tb = (NBLK2 - 1) * BLK
  g_wait(tb, rows_a, gsa)
  s_wait(rows_b, dstb_b, ssb)
  compute_blk(rows_a, tb)
  s_start(rows_a, dstb_a, ssa)
  s_wait(rows_a, dstb_a, ssa)

  plsc.subcore_barrier()
  pltpu.sync_copy(aggsh.at[pl.ds(sl, SL)], agg_hbm.at[cid, pl.ds(sl, SL)])


def _tc0_body(x_ref, w1_ref, wab_ref, o1_ref, oab_ref):
  x = x_ref[...]
  o1_ref[...] = jnp.dot(x, w1_ref[0], preferred_element_type=jnp.float32)
  oab_ref[...] = jnp.dot(x, wab_ref[...],
                         preferred_element_type=jnp.float32)


def _tc0(x, w1, wab):
  return pl.pallas_call(
      _tc0_body,
      grid=(NP // SL, 2),
      in_specs=[pl.BlockSpec((SL, D), lambda i, j: (i, 0)),
                pl.BlockSpec((1, D, DH), lambda i, j: (j, 0, 0)),
                pl.BlockSpec((D, 8), lambda i, j: (0, 0))],
      out_specs=[pl.BlockSpec((SL, DH), lambda i, j: (j * (NP // SL) + i, 0)),
                 pl.BlockSpec((SL, 8), lambda i, j: (i, 0))],
      out_shape=[jax.ShapeDtypeStruct((2 * NP, DH), jnp.float32),
                 jax.ShapeDtypeStruct((NP, 8), jnp.float32)],
  )(x, w1, wab)


def _tcdeg_body(parts_ref, a_ref, b_ref, sc_ref, o_ref):
  r = jnp.sum(parts_ref[...], axis=0)
  dego = r[0:1]
  degi = r[1:2]
  wa = sc_ref[0, 0]
  wb = sc_ref[0, 1]
  be = sc_ref[0, 2]
  apv = a_ref[...] + wa * dego + be
  bpv = b_ref[...] + wb * degi
  o_ref[...] = jnp.concatenate([apv, bpv, degi], axis=0)


def _tcdeg(parts, a2, b2, scalv):
  return pl.pallas_call(
      _tcdeg_body,
      grid=(1,),
      in_specs=[pl.BlockSpec((NW, 2, NP), lambda i: (0, 0, 0)),
                pl.BlockSpec((1, NP), lambda i: (0, 0)),
                pl.BlockSpec((1, NP), lambda i: (0, 0)),
                pl.BlockSpec((1, 128), lambda i: (0, 0))],
      out_specs=pl.BlockSpec((3, NP), lambda i: (0, 0)),
      out_shape=jax.ShapeDtypeStruct((3, NP), jnp.float32),
  )(parts, a2, b2, scalv)


def _tc1_body(p0_ref, p1_ref, dg_ref, b1_ref, w2_ref, o_ref):
  recip = 1.0 / jnp.maximum(dg_ref[...], 1.0)
  p = jnp.concatenate([p0_ref[0], p1_ref[0]], axis=1)
  pre = p * recip + b1_ref[...]
  h1 = jnp.maximum(pre, 0.0)
  o_ref[...] = jnp.dot(h1, w2_ref[...], preferred_element_type=jnp.float32)


def _tc1(agg1, dg, b1r, w2c):
  return pl.pallas_call(
      _tc1_body,
      grid=(NP // SL,),
      in_specs=[pl.BlockSpec((1, SL, DH), lambda i: (0, i, 0)),
                pl.BlockSpec((1, SL, DH), lambda i: (1, i, 0)),
                pl.BlockSpec((SL, 1), lambda i: (i, 0)),
                pl.BlockSpec((1, D), lambda i: (0, 0)),
                pl.BlockSpec((D, CP), lambda i: (0, 0))],
      out_specs=pl.BlockSpec((SL, CP), lambda i: (i, 0)),
      out_shape=jax.ShapeDtypeStruct((NP, CP), jnp.float32),
  )(agg1, agg1, dg, b1r, w2c)


def _tc2_body(q0_ref, q1_ref, dg_ref, b2_ref, o_ref):
  recip = 1.0 / jnp.maximum(dg_ref[...], 1.0)
  o_ref[...] = ((q0_ref[0] + q1_ref[0]) * recip + b2_ref[...])[:, :NCLS]


def _tc2(agg2, dg, b2r):
  nb = 10
  rb = N // nb
  return pl.pallas_call(
      _tc2_body,
      grid=(nb,),
      in_specs=[pl.BlockSpec((1, rb, CP), lambda i: (0, i, 0)),
                pl.BlockSpec((1, rb, CP), lambda i: (1, i, 0)),
                pl.BlockSpec((rb, 1), lambda i: (i, 0)),
                pl.BlockSpec((1, CP), lambda i: (0, 0))],
      out_specs=pl.BlockSpec((rb, NCLS), lambda i: (i, 0)),
      out_shape=jax.ShapeDtypeStruct((N, NCLS), jnp.float32),
  )(agg2, agg2, dg, b2r)


def kernel(h, edge_index, edge_mask, We, be, W1, b1, W2, b2):
  f32 = jnp.float32
  src = edge_index[0]
  dst = edge_index[1]
  x = jnp.pad(h, ((0, NP - N), (0, 0)))
  wab = jnp.concatenate(
      [We[:D], We[D:2 * D], jnp.zeros((D, 6), f32)], axis=1)
  w1s = jnp.stack([W1[:, :DH], W1[:, DH:]], axis=0)
  g1x, oab = _tc0(x, w1s, wab)
  a2 = oab[:, 0].reshape(1, NP)
  b2c = oab[:, 1].reshape(1, NP)
  scalv = jnp.pad(jnp.concatenate([We[2 * D:2 * D + 2, 0], be]),
                  (0, 125))[None, :]
  (parts,) = _scdeg(src, dst)
  o3 = _tcdeg(parts, a2, b2c, scalv)
  apv = o3[0]
  bpv = o3[1]
  degc = o3[2][:, None]
  mask, agg1 = _sc1(src, dst, edge_mask, apv, bpv, g1x)
  g2 = _tc1(agg1, degc, b1[None, :], jnp.pad(W2, ((0, 0), (0, CP - NCLS))))
  (agg2,) = _sc2(src, dst, mask, g2)
  return _tc2(agg2, degc, jnp.pad(b2, (0, CP - NCLS))[None, :])


# speedup vs baseline: 19.8040x; 1.0053x over previous
import functools

import jax
import jax.numpy as jnp
from jax import lax
from jax.experimental import pallas as pl
from jax.experimental.pallas import tpu as pltpu
from jax.experimental.pallas import tpu_sc as plsc

N = 10000
E = 320000
D = 128
DH = 64
NCLS = 40
CP = 48
NP = 10240
NC = 2
NS = 16
NW = NC * NS
EPS = E // NS
EPT = E // NW
BLK = 80
CHK = 4000
BPC = CHK // BLK
NCHK = EPS // CHK
NBLK2 = EPT // BLK
SL = NP // NS

_mesh = plsc.VectorSubcoreMesh(
    core_axis_name="c", subcore_axis_name="s", num_cores=NC, num_subcores=NS)

_scp = pltpu.CompilerParams(
    needs_layout_passes=False, use_tc_tiling_on_sc=False)

_GDN = lax.GatherDimensionNumbers(
    offset_dims=(), collapsed_slice_dims=(0,), start_index_map=(0,))


def _bcast_lane(v16, lane):
  idx = jnp.full((16, 1), lane, jnp.int32)
  return lax.gather(v16, idx, _GDN, (1,),
                    mode=lax.GatherScatterMode.PROMISE_IN_BOUNDS)


def _scale_rows(rows_ref, m16, gbase, nch):
  for e16 in range(16):
    mb = _bcast_lane(m16, e16)
    e = gbase + e16
    for c in range(nch):
      sl = rows_ref[e, pl.ds(c * 16, 16)]
      rows_ref[e, pl.ds(c * 16, 16)] = sl * mb


def _floop(n, body, lo=0):
  lax.fori_loop(lo, n, lambda i, c: (body(i), c)[1], 0)


@functools.partial(
    pl.kernel,
    out_type=[jax.ShapeDtypeStruct((NW, 2, NP), jnp.float32)],
    mesh=_mesh,
    compiler_params=_scp,
    scratch_types=[
        pltpu.VMEM((NP,), jnp.float32),
        pltpu.VMEM((NP,), jnp.float32),
        pltpu.VMEM((EPT,), jnp.int32),
        pltpu.VMEM((EPT,), jnp.int32),
    ],
)
def _scdeg(src_hbm, dst_hbm, parts_hbm, dego, degi, esrc, edst):
  cid = lax.axis_index("c")
  sid = lax.axis_index("s")
  wid = cid * NS + sid
  z16 = jnp.zeros((16,), jnp.float32)
  ones16 = jnp.ones((16,), jnp.float32)

  def zero_body(i):
    dego[pl.ds(i * 16, 16)] = z16
    degi[pl.ds(i * 16, 16)] = z16
  _floop(NP // 16, zero_body)

  pltpu.sync_copy(src_hbm.at[pl.ds(wid * EPT, EPT)], esrc)
  pltpu.sync_copy(dst_hbm.at[pl.ds(wid * EPT, EPT)], edst)

  def deg_body(i):
    s16 = esrc[pl.ds(i * 16, 16)]
    d16 = edst[pl.ds(i * 16, 16)]
    plsc.addupdate_scatter(dego, [s16], ones16)
    plsc.addupdate_scatter(degi, [d16], ones16)
  _floop(EPT // 16, deg_body)

  pltpu.sync_copy(dego, parts_hbm.at[wid, 0])
  pltpu.sync_copy(degi, parts_hbm.at[wid, 1])


@functools.partial(
    pl.kernel,
    out_type=[
        jax.ShapeDtypeStruct((E,), jnp.float32),
        jax.ShapeDtypeStruct((NC, NP, DH), jnp.float32),
    ],
    mesh=_mesh,
    compiler_params=_scp,
    scratch_types=[
        pltpu.VMEM((NP,), jnp.float32),
        pltpu.VMEM((NP,), jnp.float32),
        pltpu.VMEM((CHK,), jnp.int32),
        pltpu.VMEM((CHK,), jnp.int32),
        pltpu.VMEM((CHK,), jnp.float32),
        pltpu.VMEM((CHK,), jnp.float32),
        pltpu.VMEM((CHK,), jnp.int32),
        pltpu.VMEM((CHK,), jnp.int32),
        pltpu.VMEM((CHK,), jnp.float32),
        pltpu.VMEM((BLK, DH), jnp.float32),
        pltpu.VMEM((BLK, DH), jnp.float32),
        pltpu.VMEM((BLK,), jnp.int32),
        pltpu.VMEM((BLK,), jnp.int32),
        pltpu.VMEM((BLK,), jnp.int32),
        pltpu.VMEM((BLK,), jnp.int32),
        pltpu.SemaphoreType.DMA,
        pltpu.SemaphoreType.DMA,
        pltpu.SemaphoreType.DMA,
        pltpu.SemaphoreType.DMA,
        pltpu.SemaphoreType.DMA,
        pltpu.VMEM_SHARED((NP, DH), jnp.float32),
    ],
)
def _sc1(src_hbm, dst_hbm, em_hbm, ap_hbm, bp_hbm, g1x_hbm,
         mask_hbm, agg_hbm,
         ap, bp, csrc, cdst, cem, cmask, csrc2, cdst2, cem2,
         rows_a, rows_b, srcb_a, srcb_b, dstb_a, dstb_b,
         gsa, gsb, ssa, ssb, csem, aggsh):
  cid = lax.axis_index("c")
  sid = lax.axis_index("s")
  sl = sid * SL
  eb = sid * EPS
  z16 = jnp.zeros((16,), jnp.float32)
  off16 = jnp.full((16,), cid * NP, jnp.int32)

  def zero_rows(r):
    for c in range(DH // 16):
      rows_a[r, pl.ds(c * 16, 16)] = z16
  _floop(BLK, zero_rows)
  for q in range(SL // BLK):
    pltpu.sync_copy(rows_a, aggsh.at[pl.ds(sl + q * BLK, BLK)])

  pltpu.sync_copy(ap_hbm, ap)
  pltpu.sync_copy(bp_hbm, bp)
  plsc.subcore_barrier()

  def build_idx(bufs, srcb_x, dstb_x, boff):
    for g in range(BLK // 16):
      gb = g * 16
      srcb_x[pl.ds(gb, 16)] = bufs[0][pl.ds(boff + gb, 16)] + off16
      dstb_x[pl.ds(gb, 16)] = bufs[1][pl.ds(boff + gb, 16)]

  def compute_blk(bufs, rows_x, boff):
    for g in range(BLK // 16):
      gb = g * 16
      s16 = bufs[0][pl.ds(boff + gb, 16)]
      d16 = bufs[1][pl.ds(boff + gb, 16)]
      em16 = bufs[2][pl.ds(boff + gb, 16)]
      sv = plsc.load_gather(ap, [s16]) + plsc.load_gather(bp, [d16])
      m16 = em16 / (1.0 + jnp.exp(-sv))
      cmask[pl.ds(boff + gb, 16)] = m16
      _scale_rows(rows_x, m16, gb, DH // 16)

  def g_start(srcb_x, rows_x, sem):
    pltpu.async_copy(g1x_hbm.at[srcb_x], rows_x, sem)

  def g_wait(srcb_x, rows_x, sem):
    pltpu.make_async_copy(g1x_hbm.at[srcb_x], rows_x, sem).wait()

  def s_start(rows_x, dstb_x, sem):
    pltpu.async_copy(rows_x, aggsh.at[dstb_x], sem, add=True)

  def s_wait(rows_x, dstb_x, sem):
    pltpu.make_async_copy(rows_x, aggsh.at[dstb_x], sem).wait()

  sets = ((csrc, cdst, cem), (csrc2, cdst2, cem2))

  def chunk_load(c, bufs):
    ce = eb + c * CHK
    pltpu.async_copy(src_hbm.at[pl.ds(ce, CHK)], bufs[0], csem)
    pltpu.async_copy(dst_hbm.at[pl.ds(ce, CHK)], bufs[1], csem)
    pltpu.async_copy(em_hbm.at[pl.ds(ce, CHK)], bufs[2], csem)

  def chunk_wait(c, bufs):
    ce = eb + c * CHK
    pltpu.make_async_copy(src_hbm.at[pl.ds(ce, CHK)], bufs[0], csem).wait()
    pltpu.make_async_copy(dst_hbm.at[pl.ds(ce, CHK)], bufs[1], csem).wait()
    pltpu.make_async_copy(em_hbm.at[pl.ds(ce, CHK)], bufs[2], csem).wait()

  chunk_load(0, sets[0])
  for c in range(NCHK):
    ce = eb + c * CHK
    bufs = sets[c % 2]
    chunk_wait(c, bufs)
    if c + 1 < NCHK:
      chunk_load(c + 1, sets[(c + 1) % 2])
    build_idx(bufs, srcb_a, dstb_a, 0)
    g_start(srcb_a, rows_a, gsa)

    def pair(t):
      b0 = t * (2 * BLK)
      b1 = b0 + BLK
      g_wait(srcb_a, rows_a, gsa)

      @pl.when(t > 0)
      def _():
        s_wait(rows_b, dstb_b, ssb)
      build_idx(bufs, srcb_b, dstb_b, b1)
      g_start(srcb_b, rows_b, gsb)
      compute_blk(bufs, rows_a, b0)
      s_start(rows_a, dstb_a, ssa)
      g_wait(srcb_b, rows_b, gsb)

      @pl.when(t < BPC // 2 - 1)
      def _():
        s_wait(rows_a, dstb_a, ssa)
        build_idx(bufs, srcb_a, dstb_a, b1 + BLK)
        g_start(srcb_a, rows_a, gsa)
      compute_blk(bufs, rows_b, b1)
      s_start(rows_b, dstb_b, ssb)
    _floop(BPC // 2, pair)

    s_wait(rows_a, dstb_a, ssa)
    s_wait(rows_b, dstb_b, ssb)

    @pl.when(cid == 0)
    def _():
      pltpu.sync_copy(cmask, mask_hbm.at[pl.ds(ce, CHK)])

  plsc.subcore_barrier()
  pltpu.sync_copy(aggsh.at[pl.ds(sl, SL)], agg_hbm.at[cid, pl.ds(sl, SL)])


@functools.partial(
    pl.kernel,
    out_type=[jax.ShapeDtypeStruct((NC, NP, CP), jnp.float32)],
    mesh=_mesh,
    compiler_params=_scp,
    scratch_types=[
        pltpu.VMEM((EPT,), jnp.int32),
        pltpu.VMEM((EPT,), jnp.int32),
        pltpu.VMEM((EPT,), jnp.float32),
        pltpu.VMEM((BLK, CP), jnp.float32),
        pltpu.VMEM((BLK, CP), jnp.float32),
        pltpu.VMEM((BLK,), jnp.int32),
        pltpu.VMEM((BLK,), jnp.int32),
        pltpu.SemaphoreType.DMA,
        pltpu.SemaphoreType.DMA,
        pltpu.SemaphoreType.DMA,
        pltpu.SemaphoreType.DMA,
        pltpu.SemaphoreType.DMA,
        pltpu.VMEM_SHARED((NP, CP), jnp.float32),
    ],
)
def _sc2(src_hbm, dst_hbm, m_hbm, g2_hbm, agg_hbm,
         src_sw, dst_sw, m_sw, rows_a, rows_b, dstb_a, dstb_b,
         gsa, gsb, ssa, ssb, csem, aggsh):
  cid = lax.axis_index("c")
  sid = lax.axis_index("s")
  sl = sid * SL
  eb = (cid * NS + sid) * EPT
  z16 = jnp.zeros((16,), jnp.float32)

  pltpu.async_copy(src_hbm.at[pl.ds(eb, EPT)], src_sw, csem)
  pltpu.async_copy(dst_hbm.at[pl.ds(eb, EPT)], dst_sw, csem)
  pltpu.async_copy(m_hbm.at[pl.ds(eb, EPT)], m_sw, csem)

  def zero_rows(r):
    for c in range(CP // 16):
      rows_a[r, pl.ds(c * 16, 16)] = z16
  _floop(BLK, zero_rows)
  for q in range(SL // BLK):
    pltpu.sync_copy(rows_a, aggsh.at[pl.ds(sl + q * BLK, BLK)])
  plsc.subcore_barrier()

  pltpu.make_async_copy(src_hbm.at[pl.ds(eb, EPT)], src_sw, csem).wait()
  pltpu.make_async_copy(dst_hbm.at[pl.ds(eb, EPT)], dst_sw, csem).wait()
  pltpu.make_async_copy(m_hbm.at[pl.ds(eb, EPT)], m_sw, csem).wait()

  def build_idx(dstb_x, boff):
    for g in range(BLK // 16):
      gb = g * 16
      dstb_x[pl.ds(gb, 16)] = dst_sw[pl.ds(boff + gb, 16)]

  def compute_blk(rows_x, boff):
    for g in range(BLK // 16):
      gb = g * 16
      m16 = m_sw[pl.ds(boff + gb, 16)]
      _scale_rows(rows_x, m16, gb, CP // 16)

  def g_start(boff, rows_x, sem):
    pltpu.async_copy(g2_hbm.at[src_sw.at[pl.ds(boff, BLK)]], rows_x, sem)

  def g_wait(boff, rows_x, sem):
    pltpu.make_async_copy(g2_hbm.at[src_sw.at[pl.ds(boff, BLK)]],
                          rows_x, sem).wait()

  def s_start(rows_x, dstb_x, sem):
    pltpu.async_copy(rows_x, aggsh.at[dstb_x], sem, add=True)

  def s_wait(rows_x, dstb_x, sem):
    pltpu.make_async_copy(rows_x, aggsh.at[dstb_x], sem).wait()

  build_idx(dstb_a, 0)
  g_start(0, rows_a, gsa)

  def pair(t):
    b0 = t * (2 * BLK)
    b1 = b0 + BLK
    g_wait(b0, rows_a, gsa)

    @pl.when(t > 0)
    def _():
      s_wait(rows_b, dstb_b, ssb)
    build_idx(dstb_b, b1)
    g_start(b1, rows_b, gsb)
    compute_blk(rows_a, b0)
    s_start(rows_a, dstb_a, ssa)
    g_wait(b1, rows_b, gsb)

    @pl.when(t < NBLK2 // 2)
    def _():
      s_wait(rows_a, dstb_a, ssa)
      build_idx(dstb_a, b1 + BLK)
      g_start(b1 + BLK, rows_a, gsa)
    compute_blk(rows_b, b1)
    s_start(rows_b, dstb_b, ssb)
  _floop(NBLK2 // 2, pair)

  tb = (NBLK2 - 1) * BLK
  g_wait(tb, rows_a, gsa)
  s_wait(rows_b, dstb_b, ssb)
  compute_blk(rows_a, tb)
  s_start(rows_a, dstb_a, ssa)
  s_wait(rows_a, dstb_a, ssa)

  plsc.subcore_barrier()
  pltpu.sync_copy(aggsh.at[pl.ds(sl, SL)], agg_hbm.at[cid, pl.ds(sl, SL)])


def _tc0_body(x_ref, w1_ref, wab_ref, o1_ref, oab_ref):
  x = x_ref[...]
  o1_ref[...] = jnp.dot(x, w1_ref[0], preferred_element_type=jnp.float32)
  oab_ref[...] = lax.dot_general(
      wab_ref[...], x, (((0,), (1,)), ((), ())),
      preferred_element_type=jnp.float32)


def _tc0(x, w1, wab):
  return pl.pallas_call(
      _tc0_body,
      grid=(NP // SL, 2),
      in_specs=[pl.BlockSpec((SL, D), lambda i, j: (i, 0)),
                pl.BlockSpec((1, D, DH), lambda i, j: (j, 0, 0)),
                pl.BlockSpec((D, 8), lambda i, j: (0, 0))],
      out_specs=[pl.BlockSpec((SL, DH), lambda i, j: (j * (NP // SL) + i, 0)),
                 pl.BlockSpec((8, SL), lambda i, j: (0, i))],
      out_shape=[jax.ShapeDtypeStruct((2 * NP, DH), jnp.float32),
                 jax.ShapeDtypeStruct((8, NP), jnp.float32)],
  )(x, w1, wab)


def _tcdeg_body(parts_ref, ab_ref, we_ref, be_ref, o_ref, dc_ref):
  r = jnp.sum(parts_ref[...], axis=0)
  dego = r[0:1]
  degi = r[1:2]
  wa = we_ref[2 * D, 0]
  wb = we_ref[2 * D + 1, 0]
  bev = be_ref[0, 0]
  apv = ab_ref[0:1] + wa * dego + bev
  bpv = ab_ref[1:2] + wb * degi
  o_ref[...] = jnp.concatenate([apv, bpv], axis=0)
  dc_ref[...] = jnp.transpose(degi, (1, 0))


def _tcdeg(parts, oabt, we, be1):
  return pl.pallas_call(
      _tcdeg_body,
      grid=(1,),
      in_specs=[pl.BlockSpec((NW, 2, NP), lambda i: (0, 0, 0)),
                pl.BlockSpec((8, NP), lambda i: (0, 0)),
                pl.BlockSpec((2 * D + 2, 1), lambda i: (0, 0)),
                pl.BlockSpec((1, 1), lambda i: (0, 0))],
      out_specs=[pl.BlockSpec((2, NP), lambda i: (0, 0)),
                 pl.BlockSpec((NP, 1), lambda i: (0, 0))],
      out_shape=[jax.ShapeDtypeStruct((2, NP), jnp.float32),
                 jax.ShapeDtypeStruct((NP, 1), jnp.float32)],
  )(parts, oabt, we, be1)


def _tc1_body(p0_ref, p1_ref, dg_ref, b1_ref, w2_ref, o_ref):
  recip = 1.0 / jnp.maximum(dg_ref[...], 1.0)
  p = jnp.concatenate([p0_ref[0], p1_ref[0]], axis=1)
  pre = p * recip + b1_ref[...]
  h1 = jnp.maximum(pre, 0.0)
  o_ref[...] = jnp.dot(h1, w2_ref[...], preferred_element_type=jnp.float32)


def _tc1(agg1, dg, b1r, w2c):
  return pl.pallas_call(
      _tc1_body,
      grid=(NP // SL,),
      in_specs=[pl.BlockSpec((1, SL, DH), lambda i: (0, i, 0)),
                pl.BlockSpec((1, SL, DH), lambda i: (1, i, 0)),
                pl.BlockSpec((SL, 1), lambda i: (i, 0)),
                pl.BlockSpec((1, D), lambda i: (0, 0)),
                pl.BlockSpec((D, CP), lambda i: (0, 0))],
      out_specs=pl.BlockSpec((SL, CP), lambda i: (i, 0)),
      out_shape=jax.ShapeDtypeStruct((NP, CP), jnp.float32),
  )(agg1, agg1, dg, b1r, w2c)


def _tc2_body(q0_ref, q1_ref, dg_ref, b2_ref, o_ref):
  recip = 1.0 / jnp.maximum(dg_ref[...], 1.0)
  o_ref[...] = ((q0_ref[0] + q1_ref[0]) * recip + b2_ref[...])[:, :NCLS]


def _tc2(agg2, dg, b2r):
  nb = 10
  rb = N // nb
  return pl.pallas_call(
      _tc2_body,
      grid=(nb,),
      in_specs=[pl.BlockSpec((1, rb, CP), lambda i: (0, i, 0)),
                pl.BlockSpec((1, rb, CP), lambda i: (1, i, 0)),
                pl.BlockSpec((rb, 1), lambda i: (i, 0)),
                pl.BlockSpec((1, CP), lambda i: (0, 0))],
      out_specs=pl.BlockSpec((rb, NCLS), lambda i: (i, 0)),
      out_shape=jax.ShapeDtypeStruct((N, NCLS), jnp.float32),
  )(agg2, agg2, dg, b2r)


def kernel(h, edge_index, edge_mask, We, be, W1, b1, W2, b2):
  f32 = jnp.float32
  src = edge_index[0]
  dst = edge_index[1]
  x = jnp.pad(h, ((0, NP - N), (0, 0)))
  wab = jnp.concatenate(
      [We[:D], We[D:2 * D], jnp.zeros((D, 6), f32)], axis=1)
  w1s = jnp.stack([W1[:, :DH], W1[:, DH:]], axis=0)
  g1x, oabt = _tc0(x, w1s, wab)
  (parts,) = _scdeg(src, dst)
  o3, degc = _tcdeg(parts, oabt, We, be[None, :])
  apv = o3[0]
  bpv = o3[1]
  mask, agg1 = _sc1(src, dst, edge_mask, apv, bpv, g1x)
  g2 = _tc1(agg1, degc, b1[None, :], jnp.pad(W2, ((0, 0), (0, CP - NCLS))))
  (agg2,) = _sc2(src, dst, mask, g2)
  return _tc2(agg2, degc, jnp.pad(b2, (0, CP - NCLS))[None, :])


# speedup vs baseline: 19.8649x; 1.0031x over previous
import functools

import jax
import jax.numpy as jnp
from jax import lax
from jax.experimental import pallas as pl
from jax.experimental.pallas import tpu as pltpu
from jax.experimental.pallas import tpu_sc as plsc

N = 10000
E = 320000
D = 128
DH = 64
NCLS = 40
CP = 48
NP = 10240
NC = 2
NS = 16
NW = NC * NS
EPS = E // NS
EPT = E // NW
BLK = 80
CHK = 4000
BPC = CHK // BLK
NCHK = EPS // CHK
NBLK2 = EPT // BLK
SL = NP // NS

_mesh = plsc.VectorSubcoreMesh(
    core_axis_name="c", subcore_axis_name="s", num_cores=NC, num_subcores=NS)

_scp = pltpu.CompilerParams(
    needs_layout_passes=False, use_tc_tiling_on_sc=False)

_GDN = lax.GatherDimensionNumbers(
    offset_dims=(), collapsed_slice_dims=(0,), start_index_map=(0,))


def _bcast_lane(v16, lane):
  idx = jnp.full((16, 1), lane, jnp.int32)
  return lax.gather(v16, idx, _GDN, (1,),
                    mode=lax.GatherScatterMode.PROMISE_IN_BOUNDS)


def _scale_rows(rows_ref, m16, gbase, nch):
  for e16 in range(16):
    mb = _bcast_lane(m16, e16)
    e = gbase + e16
    for c in range(nch):
      sl = rows_ref[e, pl.ds(c * 16, 16)]
      rows_ref[e, pl.ds(c * 16, 16)] = sl * mb


def _floop(n, body, lo=0):
  lax.fori_loop(lo, n, lambda i, c: (body(i), c)[1], 0)


@functools.partial(
    pl.kernel,
    out_type=[jax.ShapeDtypeStruct((NW, 2, NP), jnp.float32)],
    mesh=_mesh,
    compiler_params=_scp,
    scratch_types=[
        pltpu.VMEM((NP,), jnp.float32),
        pltpu.VMEM((NP,), jnp.float32),
        pltpu.VMEM((EPT,), jnp.int32),
        pltpu.VMEM((EPT,), jnp.int32),
    ],
)
def _scdeg(src_hbm, dst_hbm, parts_hbm, dego, degi, esrc, edst):
  cid = lax.axis_index("c")
  sid = lax.axis_index("s")
  wid = cid * NS + sid
  z16 = jnp.zeros((16,), jnp.float32)
  ones16 = jnp.ones((16,), jnp.float32)

  def zero_body(i):
    dego[pl.ds(i * 16, 16)] = z16
    degi[pl.ds(i * 16, 16)] = z16
  _floop(NP // 16, zero_body)

  pltpu.sync_copy(src_hbm.at[pl.ds(wid * EPT, EPT)], esrc)
  pltpu.sync_copy(dst_hbm.at[pl.ds(wid * EPT, EPT)], edst)

  def deg_body(i):
    s16 = esrc[pl.ds(i * 16, 16)]
    d16 = edst[pl.ds(i * 16, 16)]
    plsc.addupdate_scatter(dego, [s16], ones16)
    plsc.addupdate_scatter(degi, [d16], ones16)
  _floop(EPT // 16, deg_body)

  pltpu.sync_copy(dego, parts_hbm.at[wid, 0])
  pltpu.sync_copy(degi, parts_hbm.at[wid, 1])


@functools.partial(
    pl.kernel,
    out_type=[
        jax.ShapeDtypeStruct((E,), jnp.float32),
        jax.ShapeDtypeStruct((NC, NP, DH), jnp.float32),
    ],
    mesh=_mesh,
    compiler_params=_scp,
    scratch_types=[
        pltpu.VMEM((NP,), jnp.float32),
        pltpu.VMEM((NP,), jnp.float32),
        pltpu.VMEM((CHK,), jnp.int32),
        pltpu.VMEM((CHK,), jnp.int32),
        pltpu.VMEM((CHK,), jnp.float32),
        pltpu.VMEM((CHK,), jnp.float32),
        pltpu.VMEM((CHK,), jnp.int32),
        pltpu.VMEM((CHK,), jnp.int32),
        pltpu.VMEM((CHK,), jnp.float32),
        pltpu.VMEM((BLK, DH), jnp.float32),
        pltpu.VMEM((BLK, DH), jnp.float32),
        pltpu.VMEM((BLK,), jnp.int32),
        pltpu.VMEM((BLK,), jnp.int32),
        pltpu.VMEM((BLK,), jnp.int32),
        pltpu.VMEM((BLK,), jnp.int32),
        pltpu.SemaphoreType.DMA,
        pltpu.SemaphoreType.DMA,
        pltpu.SemaphoreType.DMA,
        pltpu.SemaphoreType.DMA,
        pltpu.SemaphoreType.DMA,
        pltpu.VMEM_SHARED((NP, DH), jnp.float32),
    ],
)
def _sc1(src_hbm, dst_hbm, em_hbm, ap_hbm, bp_hbm, g1x_hbm,
         mask_hbm, agg_hbm,
         ap, bp, csrc, cdst, cem, cmask, csrc2, cdst2, cem2,
         rows_a, rows_b, srcb_a, srcb_b, dstb_a, dstb_b,
         gsa, gsb, ssa, ssb, csem, aggsh):
  cid = lax.axis_index("c")
  sid = lax.axis_index("s")
  sl = sid * SL
  eb = sid * EPS
  z16 = jnp.zeros((16,), jnp.float32)
  off16 = jnp.full((16,), cid * NP, jnp.int32)

  def zero_rows(r):
    for c in range(DH // 16):
      rows_a[r, pl.ds(c * 16, 16)] = z16
  _floop(BLK, zero_rows)
  for q in range(SL // BLK):
    pltpu.sync_copy(rows_a, aggsh.at[pl.ds(sl + q * BLK, BLK)])

  pltpu.sync_copy(ap_hbm, ap)
  pltpu.sync_copy(bp_hbm, bp)
  plsc.subcore_barrier()

  def build_idx(bufs, srcb_x, dstb_x, boff):
    for g in range(BLK // 16):
      gb = g * 16
      srcb_x[pl.ds(gb, 16)] = bufs[0][pl.ds(boff + gb, 16)] + off16
      dstb_x[pl.ds(gb, 16)] = bufs[1][pl.ds(boff + gb, 16)]

  def compute_blk(bufs, rows_x, boff):
    @plsc.parallel_loop(0, BLK // 16, unroll=BLK // 16)
    def _(g):
      gb = g * 16
      s16 = bufs[0][pl.ds(boff + gb, 16)]
      d16 = bufs[1][pl.ds(boff + gb, 16)]
      em16 = bufs[2][pl.ds(boff + gb, 16)]
      sv = plsc.load_gather(ap, [s16]) + plsc.load_gather(bp, [d16])
      m16 = em16 / (1.0 + jnp.exp(-sv))
      cmask[pl.ds(boff + gb, 16)] = m16
      _scale_rows(rows_x, m16, gb, DH // 16)

  def g_start(srcb_x, rows_x, sem):
    pltpu.async_copy(g1x_hbm.at[srcb_x], rows_x, sem)

  def g_wait(srcb_x, rows_x, sem):
    pltpu.make_async_copy(g1x_hbm.at[srcb_x], rows_x, sem).wait()

  def s_start(rows_x, dstb_x, sem):
    pltpu.async_copy(rows_x, aggsh.at[dstb_x], sem, add=True)

  def s_wait(rows_x, dstb_x, sem):
    pltpu.make_async_copy(rows_x, aggsh.at[dstb_x], sem).wait()

  sets = ((csrc, cdst, cem), (csrc2, cdst2, cem2))

  def chunk_load(c, bufs):
    ce = eb + c * CHK
    pltpu.async_copy(src_hbm.at[pl.ds(ce, CHK)], bufs[0], csem)
    pltpu.async_copy(dst_hbm.at[pl.ds(ce, CHK)], bufs[1], csem)
    pltpu.async_copy(em_hbm.at[pl.ds(ce, CHK)], bufs[2], csem)

  def chunk_wait(c, bufs):
    ce = eb + c * CHK
    pltpu.make_async_copy(src_hbm.at[pl.ds(ce, CHK)], bufs[0], csem).wait()
    pltpu.make_async_copy(dst_hbm.at[pl.ds(ce, CHK)], bufs[1], csem).wait()
    pltpu.make_async_copy(em_hbm.at[pl.ds(ce, CHK)], bufs[2], csem).wait()

  chunk_load(0, sets[0])
  for c in range(NCHK):
    ce = eb + c * CHK
    bufs = sets[c % 2]
    chunk_wait(c, bufs)
    if c + 1 < NCHK:
      chunk_load(c + 1, sets[(c + 1) % 2])
    build_idx(bufs, srcb_a, dstb_a, 0)
    g_start(srcb_a, rows_a, gsa)

    def pair(t):
      b0 = t * (2 * BLK)
      b1 = b0 + BLK
      g_wait(srcb_a, rows_a, gsa)

      @pl.when(t > 0)
      def _():
        s_wait(rows_b, dstb_b, ssb)
      build_idx(bufs, srcb_b, dstb_b, b1)
      g_start(srcb_b, rows_b, gsb)
      compute_blk(bufs, rows_a, b0)
      s_start(rows_a, dstb_a, ssa)
      g_wait(srcb_b, rows_b, gsb)

      @pl.when(t < BPC // 2 - 1)
      def _():
        s_wait(rows_a, dstb_a, ssa)
        build_idx(bufs, srcb_a, dstb_a, b1 + BLK)
        g_start(srcb_a, rows_a, gsa)
      compute_blk(bufs, rows_b, b1)
      s_start(rows_b, dstb_b, ssb)
    _floop(BPC // 2, pair)

    s_wait(rows_a, dstb_a, ssa)
    s_wait(rows_b, dstb_b, ssb)

    @pl.when(cid == 0)
    def _():
      pltpu.sync_copy(cmask, mask_hbm.at[pl.ds(ce, CHK)])

  plsc.subcore_barrier()
  pltpu.sync_copy(aggsh.at[pl.ds(sl, SL)], agg_hbm.at[cid, pl.ds(sl, SL)])


@functools.partial(
    pl.kernel,
    out_type=[jax.ShapeDtypeStruct((NC, NP, CP), jnp.float32)],
    mesh=_mesh,
    compiler_params=_scp,
    scratch_types=[
        pltpu.VMEM((EPT,), jnp.int32),
        pltpu.VMEM((EPT,), jnp.int32),
        pltpu.VMEM((EPT,), jnp.float32),
        pltpu.VMEM((BLK, CP), jnp.float32),
        pltpu.VMEM((BLK, CP), jnp.float32),
        pltpu.VMEM((BLK,), jnp.int32),
        pltpu.VMEM((BLK,), jnp.int32),
        pltpu.SemaphoreType.DMA,
        pltpu.SemaphoreType.DMA,
        pltpu.SemaphoreType.DMA,
        pltpu.SemaphoreType.DMA,
        pltpu.SemaphoreType.DMA,
        pltpu.VMEM_SHARED((NP, CP), jnp.float32),
    ],
)
def _sc2(src_hbm, dst_hbm, m_hbm, g2_hbm, agg_hbm,
         src_sw, dst_sw, m_sw, rows_a, rows_b, dstb_a, dstb_b,
         gsa, gsb, ssa, ssb, csem, aggsh):
  cid = lax.axis_index("c")
  sid = lax.axis_index("s")
  sl = sid * SL
  eb = (cid * NS + sid) * EPT
  z16 = jnp.zeros((16,), jnp.float32)

  pltpu.async_copy(src_hbm.at[pl.ds(eb, EPT)], src_sw, csem)
  pltpu.async_copy(dst_hbm.at[pl.ds(eb, EPT)], dst_sw, csem)
  pltpu.async_copy(m_hbm.at[pl.ds(eb, EPT)], m_sw, csem)

  def zero_rows(r):
    for c in range(CP // 16):
      rows_a[r, pl.ds(c * 16, 16)] = z16
  _floop(BLK, zero_rows)
  for q in range(SL // BLK):
    pltpu.sync_copy(rows_a, aggsh.at[pl.ds(sl + q * BLK, BLK)])
  plsc.subcore_barrier()

  pltpu.make_async_copy(src_hbm.at[pl.ds(eb, EPT)], src_sw, csem).wait()
  pltpu.make_async_copy(dst_hbm.at[pl.ds(eb, EPT)], dst_sw, csem).wait()
  pltpu.make_async_copy(m_hbm.at[pl.ds(eb, EPT)], m_sw, csem).wait()

  def build_idx(dstb_x, boff):
    for g in range(BLK // 16):
      gb = g * 16
      dstb_x[pl.ds(gb, 16)] = dst_sw[pl.ds(boff + gb, 16)]

  def compute_blk(rows_x, boff):
    @plsc.parallel_loop(0, BLK // 16, unroll=BLK // 16)
    def _(g):
      gb = g * 16
      m16 = m_sw[pl.ds(boff + gb, 16)]
      _scale_rows(rows_x, m16, gb, CP // 16)

  def g_start(boff, rows_x, sem):
    pltpu.async_copy(g2_hbm.at[src_sw.at[pl.ds(boff, BLK)]], rows_x, sem)

  def g_wait(boff, rows_x, sem):
    pltpu.make_async_copy(g2_hbm.at[src_sw.at[pl.ds(boff, BLK)]],
                          rows_x, sem).wait()

  def s_start(rows_x, dstb_x, sem):
    pltpu.async_copy(rows_x, aggsh.at[dstb_x], sem, add=True)

  def s_wait(rows_x, dstb_x, sem):
    pltpu.make_async_copy(rows_x, aggsh.at[dstb_x], sem).wait()

  build_idx(dstb_a, 0)
  g_start(0, rows_a, gsa)

  def pair(t):
    b0 = t * (2 * BLK)
    b1 = b0 + BLK
    g_wait(b0, rows_a, gsa)

    @pl.when(t > 0)
    def _():
      s_wait(rows_b, dstb_b, ssb)
    build_idx(dstb_b, b1)
    g_start(b1, rows_b, gsb)
    compute_blk(rows_a, b0)
    s_start(rows_a, dstb_a, ssa)
    g_wait(b1, rows_b, gsb)

    @pl.when(t < NBLK2 // 2)
    def _():
      s_wait(rows_a, dstb_a, ssa)
      build_idx(dstb_a, b1 + BLK)
      g_start(b1 + BLK, rows_a, gsa)
    compute_blk(rows_b, b1)
    s_start(rows_b, dstb_b, ssb)
  _floop(NBLK2 // 2, pair)

  tb = (NBLK2 - 1) * BLK
  g_wait(tb, rows_a, gsa)
  s_wait(rows_b, dstb_b, ssb)
  compute_blk(rows_a, tb)
  s_start(rows_a, dstb_a, ssa)
  s_wait(rows_a, dstb_a, ssa)

  plsc.subcore_barrier()
  pltpu.sync_copy(aggsh.at[pl.ds(sl, SL)], agg_hbm.at[cid, pl.ds(sl, SL)])


def _tc0_body(x_ref, w1_ref, wab_ref, o1_ref, oab_ref):
  x = x_ref[...]
  o1_ref[...] = jnp.dot(x, w1_ref[0], preferred_element_type=jnp.float32)
  oab_ref[...] = lax.dot_general(
      wab_ref[...], x, (((0,), (1,)), ((), ())),
      preferred_element_type=jnp.float32)


def _tc0(x, w1, wab):
  return pl.pallas_call(
      _tc0_body,
      grid=(NP // SL, 2),
      in_specs=[pl.BlockSpec((SL, D), lambda i, j: (i, 0)),
                pl.BlockSpec((1, D, DH), lambda i, j: (j, 0, 0)),
                pl.BlockSpec((D, 8), lambda i, j: (0, 0))],
      out_specs=[pl.BlockSpec((SL, DH), lambda i, j: (j * (NP // SL) + i, 0)),
                 pl.BlockSpec((8, SL), lambda i, j: (0, i))],
      out_shape=[jax.ShapeDtypeStruct((2 * NP, DH), jnp.float32),
                 jax.ShapeDtypeStruct((8, NP), jnp.float32)],
  )(x, w1, wab)


def _tcdeg_body(parts_ref, ab_ref, we_ref, be_ref, o_ref, dc_ref):
  r = jnp.sum(parts_ref[...], axis=0)
  dego = r[0:1]
  degi = r[1:2]
  wa = we_ref[2 * D, 0]
  wb = we_ref[2 * D + 1, 0]
  bev = be_ref[0, 0]
  apv = ab_ref[0:1] + wa * dego + bev
  bpv = ab_ref[1:2] + wb * degi
  o_ref[...] = jnp.concatenate([apv, bpv], axis=0)
  dc_ref[...] = jnp.transpose(degi, (1, 0))


def _tcdeg(parts, oabt, we, be1):
  return pl.pallas_call(
      _tcdeg_body,
      grid=(1,),
      in_specs=[pl.BlockSpec((NW, 2, NP), lambda i: (0, 0, 0)),
                pl.BlockSpec((8, NP), lambda i: (0, 0)),
                pl.BlockSpec((2 * D + 2, 1), lambda i: (0, 0)),
                pl.BlockSpec((1, 1), lambda i: (0, 0))],
      out_specs=[pl.BlockSpec((2, NP), lambda i: (0, 0)),
                 pl.BlockSpec((NP, 1), lambda i: (0, 0))],
      out_shape=[jax.ShapeDtypeStruct((2, NP), jnp.float32),
                 jax.ShapeDtypeStruct((NP, 1), jnp.float32)],
  )(parts, oabt, we, be1)


def _tc1_body(p0_ref, p1_ref, dg_ref, b1_ref, w2_ref, o_ref):
  recip = 1.0 / jnp.maximum(dg_ref[...], 1.0)
  p = jnp.concatenate([p0_ref[0], p1_ref[0]], axis=1)
  pre = p * recip + b1_ref[...]
  h1 = jnp.maximum(pre, 0.0)
  o_ref[...] = jnp.dot(h1, w2_ref[...], preferred_element_type=jnp.float32)


def _tc1(agg1, dg, b1r, w2c):
  return pl.pallas_call(
      _tc1_body,
      grid=(NP // SL,),
      in_specs=[pl.BlockSpec((1, SL, DH), lambda i: (0, i, 0)),
                pl.BlockSpec((1, SL, DH), lambda i: (1, i, 0)),
                pl.BlockSpec((SL, 1), lambda i: (i, 0)),
                pl.BlockSpec((1, D), lambda i: (0, 0)),
                pl.BlockSpec((D, CP), lambda i: (0, 0))],
      out_specs=pl.BlockSpec((SL, CP), lambda i: (i, 0)),
      out_shape=jax.ShapeDtypeStruct((NP, CP), jnp.float32),
  )(agg1, agg1, dg, b1r, w2c)


def _tc2_body(q0_ref, q1_ref, dg_ref, b2_ref, o_ref):
  recip = 1.0 / jnp.maximum(dg_ref[...], 1.0)
  o_ref[...] = ((q0_ref[0] + q1_ref[0]) * recip + b2_ref[...])[:, :NCLS]


def _tc2(agg2, dg, b2r):
  nb = 10
  rb = N // nb
  return pl.pallas_call(
      _tc2_body,
      grid=(nb,),
      in_specs=[pl.BlockSpec((1, rb, CP), lambda i: (0, i, 0)),
                pl.BlockSpec((1, rb, CP), lambda i: (1, i, 0)),
                pl.BlockSpec((rb, 1), lambda i: (i, 0)),
                pl.BlockSpec((1, CP), lambda i: (0, 0))],
      out_specs=pl.BlockSpec((rb, NCLS), lambda i: (i, 0)),
      out_shape=jax.ShapeDtypeStruct((N, NCLS), jnp.float32),
  )(agg2, agg2, dg, b2r)


def kernel(h, edge_index, edge_mask, We, be, W1, b1, W2, b2):
  f32 = jnp.float32
  src = edge_index[0]
  dst = edge_index[1]
  x = jnp.pad(h, ((0, NP - N), (0, 0)))
  wab = jnp.concatenate(
      [We[:D], We[D:2 * D], jnp.zeros((D, 6), f32)], axis=1)
  w1s = jnp.stack([W1[:, :DH], W1[:, DH:]], axis=0)
  g1x, oabt = _tc0(x, w1s, wab)
  (parts,) = _scdeg(src, dst)
  o3, degc = _tcdeg(parts, oabt, We, be[None, :])
  apv = o3[0]
  bpv = o3[1]
  mask, agg1 = _sc1(src, dst, edge_mask, apv, bpv, g1x)
  g2 = _tc1(agg1, degc, b1[None, :], jnp.pad(W2, ((0, 0), (0, CP - NCLS))))
  (agg2,) = _sc2(src, dst, mask, g2)
  return _tc2(agg2, degc, jnp.pad(b2, (0, CP - NCLS))[None, :])
